# baseline (speedup 1.0000x reference)
import functools

import numpy as np
import jax
import jax.numpy as jnp
from jax import lax
from jax.experimental import pallas as pl
from jax.experimental.pallas import tpu as pltpu
from jax.experimental.pallas import tpu_sc as plsc

B = 2
S = 1024
D = 1024
H = 16
DH = D // H
E = 16
DFF = 2048
T = B * S
CAP = int(1.2 * T / E)
CPAD = 160
VOCAB = 32000
BT = 256
NT = T // BT
VT = 1280
NV = VOCAB // VT
DC = 512
_NW = 32

_HI = lax.Precision.HIGHEST


def _dot_nt(a, b):
  return lax.dot_general(a, b, (((1,), (1,)), ((), ())), precision=_HI,
                         preferred_element_type=jnp.float32)


def _dot_nn(a, b):
  return lax.dot_general(a, b, (((1,), (0,)), ((), ())), precision=_HI,
                         preferred_element_type=jnp.float32)


def _ln(x, g, b):
  m = jnp.mean(x, axis=-1, keepdims=True)
  v = jnp.mean((x - m) ** 2, axis=-1, keepdims=True)
  return (x - m) / jnp.sqrt(v + 1e-5) * g + b


def _pe_np():
  pos = np.arange(S, dtype=np.float32)[:, None]
  div = np.exp(np.arange(0, D, 2, dtype=np.float32) * (-np.log(10000.0) / D))
  pe = np.zeros((S, D), dtype=np.float32)
  pe[:, 0::2] = np.sin(pos * div)
  pe[:, 1::2] = np.cos(pos * div)
  return pe


def _sc_gather_rows(table, idx):
  t = idx.shape[0]
  d = table.shape[1]
  bpw = t // _NW
  idx2 = idx.reshape(_NW, bpw)
  mesh = plsc.VectorSubcoreMesh(core_axis_name="c", subcore_axis_name="s")

  @functools.partial(
      pl.kernel, mesh=mesh,
      out_type=jax.ShapeDtypeStruct((t, d), jnp.float32),
      scratch_types=[
          pltpu.VMEM((bpw,), jnp.int32),
          pltpu.VMEM((bpw, d), jnp.float32),
          pltpu.SemaphoreType.DMA,
      ],
  )
  def k(table_hbm, idx_hbm, out_hbm, idx_v, rows_v, sem):
    wid = lax.axis_index("s") * 2 + lax.axis_index("c")
    pltpu.sync_copy(idx_hbm.at[wid], idx_v)
    pltpu.async_copy(table_hbm.at[idx_v], rows_v, sem).wait()
    pltpu.sync_copy(rows_v, out_hbm.at[pl.ds(wid * bpw, bpw)])

  return k(table, idx2)


def _sc_scatter_rows(rows, dst, n_out):
  t, d = rows.shape
  bpw = t // _NW
  dst2 = dst.reshape(_NW, bpw)
  mesh = plsc.VectorSubcoreMesh(core_axis_name="c", subcore_axis_name="s")

  @functools.partial(
      pl.kernel, mesh=mesh,
      out_type=jax.ShapeDtypeStruct((n_out, d), jnp.float32),
      scratch_types=[
          pltpu.VMEM((bpw,), jnp.int32),
          pltpu.VMEM((bpw, d), jnp.float32),
          pltpu.SemaphoreType.DMA,
      ],
  )
  def k(rows_hbm, dst_hbm, out_hbm, idx_v, rows_v, sem):
    wid = lax.axis_index("s") * 2 + lax.axis_index("c")
    pltpu.sync_copy(dst_hbm.at[wid], idx_v)
    pltpu.sync_copy(rows_hbm.at[pl.ds(wid * bpw, bpw)], rows_v)
    pltpu.async_copy(rows_v, out_hbm.at[idx_v], sem).wait()

  return k(rows, dst2)


def _k_addpe(emb, pe):
  def body(e_ref, p_ref, o_ref):
    o_ref[...] = e_ref[...] + p_ref[...]

  return pl.pallas_call(
      body,
      grid=(NT,),
      in_specs=[pl.BlockSpec((BT, D), lambda i: (i, 0)),
                pl.BlockSpec((BT, D), lambda i: (i % (S // BT), 0))],
      out_specs=pl.BlockSpec((BT, D), lambda i: (i, 0)),
      out_shape=jax.ShapeDtypeStruct((T, D), jnp.float32),
  )(emb, pe)


def _k_ln_qkv(x, g, b, w, wb):
  def body(x_ref, g_ref, b_ref, w_ref, wb_ref, o_ref):
    z = _ln(x_ref[...], g_ref[...], b_ref[...])
    o_ref[...] = _dot_nt(z, w_ref[...]) + wb_ref[...]

  return pl.pallas_call(
      body,
      grid=(3, NT),
      in_specs=[
          pl.BlockSpec((BT, D), lambda j, t: (t, 0)),
          pl.BlockSpec((1, D), lambda j, t: (0, 0)),
          pl.BlockSpec((1, D), lambda j, t: (0, 0)),
          pl.BlockSpec((D, D), lambda j, t: (j, 0)),
          pl.BlockSpec((1, D), lambda j, t: (0, j)),
      ],
      out_specs=pl.BlockSpec((BT, D), lambda j, t: (t, j)),
      out_shape=jax.ShapeDtypeStruct((T, 3 * D), jnp.float32),
  )(x, g.reshape(1, D), b.reshape(1, D), w, wb.reshape(1, 3 * D))


def _k_attn(qkv):
  scale = 1.0 / float(np.sqrt(DH))
  nh2 = H // 2

  def body(q_ref, k_ref, v_ref, o_ref):
    outs = []
    for u in range(2):
      q = q_ref[:, u * DH:(u + 1) * DH]
      kk = k_ref[:, u * DH:(u + 1) * DH]
      v = v_ref[:, u * DH:(u + 1) * DH]
      s_mat = _dot_nt(q, kk) * scale
      m = jnp.max(s_mat, axis=1, keepdims=True)
      p = jnp.exp(s_mat - m)
      l = jnp.sum(p, axis=1, keepdims=True)
      outs.append(_dot_nn(p / l, v))
    o_ref[...] = jnp.concatenate(outs, axis=1)

  return pl.pallas_call(
      body,
      grid=(B, nh2),
      in_specs=[
          pl.BlockSpec((S, 2 * DH), lambda b, h: (b, h)),
          pl.BlockSpec((S, 2 * DH), lambda b, h: (b, nh2 + h)),
          pl.BlockSpec((S, 2 * DH), lambda b, h: (b, 2 * nh2 + h)),
      ],
      out_specs=pl.BlockSpec((S, 2 * DH), lambda b, h: (b, h)),
      out_shape=jax.ShapeDtypeStruct((T, D), jnp.float32),
  )(qkv, qkv, qkv)


def _k_proj_res(x, ao, w, wb):
  def body(x_ref, a_ref, w_ref, b_ref, o_ref):
    o_ref[...] = x_ref[...] + _dot_nt(a_ref[...], w_ref[...]) + b_ref[...]

  return pl.pallas_call(
      body,
      grid=(NT,),
      in_specs=[
          pl.BlockSpec((BT, D), lambda t: (t, 0)),
          pl.BlockSpec((BT, D), lambda t: (t, 0)),
          pl.BlockSpec((D, D), lambda t: (0, 0)),
          pl.BlockSpec((1, D), lambda t: (0, 0)),
      ],
      out_specs=pl.BlockSpec((BT, D), lambda t: (t, 0)),
      out_shape=jax.ShapeDtypeStruct((T, D), jnp.float32),
  )(x, ao, w, wb.reshape(1, D))


def _k_ln2_router(x, g, b, sw, sb):
  def body(x_ref, g_ref, b_ref, sw_ref, sb_ref, z_ref, rp_ref, rpm_ref, rt_ref):
    z = _ln(x_ref[...], g_ref[...], b_ref[...])
    z_ref[...] = z
    lg = _dot_nn(z, sw_ref[...]) + sb_ref[...]
    mx = jnp.max(lg, axis=1, keepdims=True)
    ex = jnp.exp(lg - mx)
    rp = ex / jnp.sum(ex, axis=1, keepdims=True)
    rp_ref[...] = rp
    pm = jnp.max(rp, axis=1, keepdims=True)
    rpm_ref[...] = pm
    ii = lax.broadcasted_iota(jnp.int32, (BT, E), 1)
    rt_ref[...] = jnp.min(jnp.where(rp == pm, ii, E), axis=1, keepdims=True)

  return pl.pallas_call(
      body,
      grid=(NT,),
      in_specs=[
          pl.BlockSpec((BT, D), lambda t: (t, 0)),
          pl.BlockSpec((1, D), lambda t: (0, 0)),
          pl.BlockSpec((1, D), lambda t: (0, 0)),
          pl.BlockSpec((D, E), lambda t: (0, 0)),
          pl.BlockSpec((1, E), lambda t: (0, 0)),
      ],
      out_specs=[
          pl.BlockSpec((BT, D), lambda t: (t, 0)),
          pl.BlockSpec((BT, E), lambda t: (t, 0)),
          pl.BlockSpec((BT, 1), lambda t: (t, 0)),
          pl.BlockSpec((BT, 1), lambda t: (t, 0)),
      ],
      out_shape=[
          jax.ShapeDtypeStruct((T, D), jnp.float32),
          jax.ShapeDtypeStruct((T, E), jnp.float32),
          jax.ShapeDtypeStruct((T, 1), jnp.float32),
          jax.ShapeDtypeStruct((T, 1), jnp.int32),
      ],
  )(x, g.reshape(1, D), b.reshape(1, D), sw, sb.reshape(1, E))


def _k_route(rt_s, rp_s):
  def body(rt_ref, rp_ref, dst_ref, gi_ref, kp_ref, cnt_ref, ps_ref, nd_ref):
    rt = rt_ref[...]
    oh = (rt == lax.broadcasted_iota(jnp.int32, (T, E), 1)).astype(jnp.int32)
    c = oh
    sh = 1
    while sh < T:
      c = c + jnp.concatenate(
          [jnp.zeros((sh, E), jnp.int32), c[:T - sh]], axis=0)
      sh *= 2
    pos = jnp.sum(c * oh, axis=1, keepdims=True) - 1
    keep = (pos < CAP).astype(jnp.int32)
    cnt_ref[...] = c[T - 1:T, :].astype(jnp.float32)
    ps_ref[...] = jnp.sum(rp_ref[...], axis=0, keepdims=True)
    nd_ref[...] = jnp.sum(1 - keep, axis=0, keepdims=True)
    dst_ref[...] = rt * CPAD + jnp.minimum(pos, CAP)
    gi_ref[...] = rt * CPAD + jnp.minimum(pos, CAP - 1)
    kp_ref[...] = keep

  return pl.pallas_call(
      body,
      out_shape=[
          jax.ShapeDtypeStruct((T, 1), jnp.int32),
          jax.ShapeDtypeStruct((T, 1), jnp.int32),
          jax.ShapeDtypeStruct((T, 1), jnp.int32),
          jax.ShapeDtypeStruct((1, E), jnp.float32),
          jax.ShapeDtypeStruct((1, E), jnp.float32),
          jax.ShapeDtypeStruct((1, 1), jnp.int32),
      ],
  )(rt_s, rp_s)


def _k_expert(buf, w1, b1, w2, b2):
  def body(buf_ref, w1_ref, b1_ref, w2_ref, b2_ref, o_ref):
    j = pl.program_id(1)
    h = jnp.maximum(_dot_nn(buf_ref[...], w1_ref[0]) + b1_ref[0], 0.0)
    part = _dot_nn(h, w2_ref[0])

    @pl.when(j == 0)
    def _():
      o_ref[...] = part + b2_ref[0]

    @pl.when(j != 0)
    def _():
      o_ref[...] = o_ref[...] + part

  return pl.pallas_call(
      body,
      grid=(E, DFF // DC),
      in_specs=[
          pl.BlockSpec((CPAD, D), lambda e, j: (e, 0)),
          pl.BlockSpec((1, D, DC), lambda e, j: (e, 0, j)),
          pl.BlockSpec((1, 1, DC), lambda e, j: (e, 0, j)),
          pl.BlockSpec((1, DC, D), lambda e, j: (e, j, 0)),
          pl.BlockSpec((1, 1, D), lambda e, j: (e, 0, 0)),
      ],
      out_specs=pl.BlockSpec((CPAD, D), lambda e, j: (e, 0)),
      out_shape=jax.ShapeDtypeStruct((E * CPAD, D), jnp.float32),
  )(buf, w1, b1.reshape(E, 1, DFF), w2, b2.reshape(E, 1, D))


def _k_combine(x1, z, gat, kp, rpm):
  def body(x_ref, z_ref, g_ref, k_ref, p_ref, o_ref):
    sel = jnp.where(k_ref[...] != 0, g_ref[...], z_ref[...])
    o_ref[...] = x_ref[...] + sel * p_ref[...]

  return pl.pallas_call(
      body,
      grid=(NT,),
      in_specs=[
          pl.BlockSpec((BT, D), lambda t: (t, 0)),
          pl.BlockSpec((BT, D), lambda t: (t, 0)),
          pl.BlockSpec((BT, D), lambda t: (t, 0)),
          pl.BlockSpec((BT, 1), lambda t: (t, 0)),
          pl.BlockSpec((BT, 1), lambda t: (t, 0)),
      ],
      out_specs=pl.BlockSpec((BT, D), lambda t: (t, 0)),
      out_shape=jax.ShapeDtypeStruct((T, D), jnp.float32),
  )(x1, z, gat, kp, rpm)


def _k_fln(x, g, b):
  def body(x_ref, g_ref, b_ref, o_ref):
    o_ref[...] = _ln(x_ref[...], g_ref[...], b_ref[...]).astype(jnp.bfloat16)

  return pl.pallas_call(
      body,
      grid=(NT,),
      in_specs=[
          pl.BlockSpec((BT, D), lambda t: (t, 0)),
          pl.BlockSpec((1, D), lambda t: (0, 0)),
          pl.BlockSpec((1, D), lambda t: (0, 0)),
      ],
      out_specs=pl.BlockSpec((BT, D), lambda t: (t, 0)),
      out_shape=jax.ShapeDtypeStruct((T, D), jnp.bfloat16),
  )(x, g.reshape(1, D), b.reshape(1, D))


def _k_dec_lse(xb, w, db):
  def body(x_ref, w_ref, db_ref, lse_ref, m_acc, s_acc):
    v = pl.program_id(0)

    @pl.when(v == 0)
    def _():
      m_acc[...] = jnp.full((T, 1), -1e30, jnp.float32)
      s_acc[...] = jnp.zeros((T, 1), jnp.float32)

    wb = w_ref[...].astype(jnp.bfloat16)
    for c in range(NT):
      xc = x_ref[pl.ds(c * BT, BT), :]
      lg = lax.dot_general(xc, wb, (((1,), (1,)), ((), ())),
                           preferred_element_type=jnp.float32) + db_ref[...]
      mo = m_acc[pl.ds(c * BT, BT), :]
      so = s_acc[pl.ds(c * BT, BT), :]
      tm = jnp.max(lg, axis=1, keepdims=True)
      mn = jnp.maximum(mo, tm)
      sn = so * jnp.exp(mo - mn) + jnp.sum(jnp.exp(lg - mn), axis=1,
                                           keepdims=True)
      m_acc[pl.ds(c * BT, BT), :] = mn
      s_acc[pl.ds(c * BT, BT), :] = sn

    @pl.when(v == NV - 1)
    def _():
      lse_ref[...] = m_acc[...] + jnp.log(s_acc[...])

  return pl.pallas_call(
      body,
      grid=(NV,),
      in_specs=[
          pl.BlockSpec((T, D), lambda v: (0, 0)),
          pl.BlockSpec((VT, D), lambda v: (v, 0)),
          pl.BlockSpec((1, VT), lambda v: (0, v)),
      ],
      out_specs=pl.BlockSpec((T, 1), lambda v: (0, 0)),
      out_shape=jax.ShapeDtypeStruct((T, 1), jnp.float32),
      scratch_shapes=[pltpu.VMEM((T, 1), jnp.float32),
                      pltpu.VMEM((T, 1), jnp.float32)],
  )(xb, w, db)


def _k_dec_out(xb, w, db, lse):
  def body(x_ref, w_ref, db_ref, l_ref, o_ref, wb_s):
    t = pl.program_id(1)

    @pl.when(t == 0)
    def _():
      wb_s[...] = w_ref[...].astype(jnp.bfloat16)

    xc = x_ref[pl.ds(t * BT, BT), :]
    lg = lax.dot_general(xc, wb_s[...], (((1,), (1,)), ((), ())),
                         preferred_element_type=jnp.float32) + db_ref[...]
    o_ref[...] = lg - l_ref[pl.ds(t * BT, BT), :]

  return pl.pallas_call(
      body,
      grid=(NV, NT),
      in_specs=[
          pl.BlockSpec((T, D), lambda v, t: (0, 0)),
          pl.BlockSpec((VT, D), lambda v, t: (v, 0)),
          pl.BlockSpec((1, VT), lambda v, t: (0, v)),
          pl.BlockSpec((T, 1), lambda v, t: (0, 0)),
      ],
      out_specs=pl.BlockSpec((BT, VT), lambda v, t: (t, v)),
      out_shape=jax.ShapeDtypeStruct((T, VOCAB), jnp.float32),
      scratch_shapes=[pltpu.VMEM((VT, D), jnp.bfloat16)],
  )(xb, w, db, lse)


def _to_s(a):
  return a.reshape(B, S, -1).transpose(1, 0, 2).reshape(T, -1)


def _to_b(a):
  return a.reshape(S, B, -1).transpose(1, 0, 2).reshape(T, -1)


def kernel(input_chars, embed_w, ln1_g, ln1_b, ln2_g, ln2_b, attn_in_w,
           attn_in_b, attn_out_w, attn_out_b, switch_w, switch_b, exp_w1,
           exp_b1, exp_w2, exp_b2, fln_g, fln_b, dec_w, dec_b):
  idx = input_chars.reshape(T)
  emb = _sc_gather_rows(embed_w, idx)
  x = _k_addpe(emb, jnp.asarray(_pe_np()))
  counts_l, ps_l, nd_l, rpm_l = [], [], [], []
  for i in range(2):
    qkv = _k_ln_qkv(x, ln1_g[i], ln1_b[i], attn_in_w[i], attn_in_b[i])
    ao = _k_attn(qkv)
    x1 = _k_proj_res(x, ao, attn_out_w[i], attn_out_b[i])
    z, rp, rpm, rt = _k_ln2_router(x1, ln2_g[i], ln2_b[i], switch_w[i],
                                   switch_b[i])
    dst_s, gi_s, kp_s, cnt, ps, nd = _k_route(_to_s(rt), _to_s(rp))
    dst = _to_b(dst_s).reshape(T)
    gi = _to_b(gi_s).reshape(T)
    kp = _to_b(kp_s)
    buf = _sc_scatter_rows(z, dst, E * CPAD)
    eo = _k_expert(buf, exp_w1[i], exp_b1[i], exp_w2[i], exp_b2[i])
    gat = _sc_gather_rows(eo, gi)
    x = _k_combine(x1, z, gat, kp, rpm)
    counts_l.append(cnt.reshape(E))
    ps_l.append(ps.reshape(E))
    nd_l.append(nd.reshape(()))
    rpm_l.append(_to_s(rpm).reshape(T))
  xb = _k_fln(x, fln_g, fln_b)
  db2 = dec_b.reshape(1, VOCAB)
  lse = _k_dec_lse(xb, dec_w, db2)
  lp = _k_dec_out(xb, dec_w, db2, lse)
  return (lp.reshape(B, S, VOCAB), jnp.stack(counts_l), jnp.stack(ps_l),
          jnp.stack(nd_l), jnp.stack(rpm_l))


# speedup vs baseline: 1.3067x; 1.3067x over previous
import functools

import numpy as np
import jax
import jax.numpy as jnp
from jax import lax
from jax.experimental import pallas as pl
from jax.experimental.pallas import tpu as pltpu
from jax.experimental.pallas import tpu_sc as plsc

B = 2
S = 1024
D = 1024
H = 16
DH = D // H
E = 16
DFF = 2048
T = B * S
CAP = int(1.2 * T / E)
CPAD = 160
VOCAB = 32000
BT = 256
NT = T // BT
VT = 3200
NV = VOCAB // VT
DC = 512
_NW = 32

def _split_hl(a):
  ah = a.astype(jnp.bfloat16)
  al = (a - ah.astype(jnp.float32)).astype(jnp.bfloat16)
  return ah, al


def _dot_nt(a, b, precision=None):
  dims = (((1,), (1,)), ((), ()))
  if precision is not None:
    return lax.dot_general(a, b, dims, precision=precision,
                           preferred_element_type=jnp.float32)
  ah, al = _split_hl(a)
  bh, bl = _split_hl(b)
  d = lambda x, y: lax.dot_general(x, y, dims,
                                   preferred_element_type=jnp.float32)
  return d(ah, bh) + (d(ah, bl) + d(al, bh))


def _dot_nn(a, b, precision=None):
  dims = (((1,), (0,)), ((), ()))
  if precision is not None:
    return lax.dot_general(a, b, dims, precision=precision,
                           preferred_element_type=jnp.float32)
  ah, al = _split_hl(a)
  bh, bl = _split_hl(b)
  d = lambda x, y: lax.dot_general(x, y, dims,
                                   preferred_element_type=jnp.float32)
  return d(ah, bh) + (d(ah, bl) + d(al, bh))


def _ln(x, g, b):
  m = jnp.mean(x, axis=-1, keepdims=True)
  v = jnp.mean((x - m) ** 2, axis=-1, keepdims=True)
  return (x - m) / jnp.sqrt(v + 1e-5) * g + b


def _pe_np():
  pos = np.arange(S, dtype=np.float32)[:, None]
  div = np.exp(np.arange(0, D, 2, dtype=np.float32) * (-np.log(10000.0) / D))
  pe = np.zeros((S, D), dtype=np.float32)
  pe[:, 0::2] = np.sin(pos * div)
  pe[:, 1::2] = np.cos(pos * div)
  return pe


def _sc_gather_rows(table, idx):
  t = idx.shape[0]
  d = table.shape[1]
  bpw = t // _NW
  idx2 = idx.reshape(_NW, bpw)
  mesh = plsc.VectorSubcoreMesh(core_axis_name="c", subcore_axis_name="s")

  @functools.partial(
      pl.kernel, mesh=mesh,
      out_type=jax.ShapeDtypeStruct((t, d), jnp.float32),
      scratch_types=[
          pltpu.VMEM((bpw,), jnp.int32),
          pltpu.VMEM((bpw, d), jnp.float32),
          pltpu.SemaphoreType.DMA,
      ],
  )
  def k(table_hbm, idx_hbm, out_hbm, idx_v, rows_v, sem):
    wid = lax.axis_index("s") * 2 + lax.axis_index("c")
    pltpu.sync_copy(idx_hbm.at[wid], idx_v)
    pltpu.async_copy(table_hbm.at[idx_v], rows_v, sem).wait()
    pltpu.sync_copy(rows_v, out_hbm.at[pl.ds(wid * bpw, bpw)])

  return k(table, idx2)


def _sc_scatter_rows(rows, dst, n_out):
  t, d = rows.shape
  bpw = t // _NW
  dst2 = dst.reshape(_NW, bpw)
  mesh = plsc.VectorSubcoreMesh(core_axis_name="c", subcore_axis_name="s")

  @functools.partial(
      pl.kernel, mesh=mesh,
      out_type=jax.ShapeDtypeStruct((n_out, d), jnp.float32),
      scratch_types=[
          pltpu.VMEM((bpw,), jnp.int32),
          pltpu.VMEM((bpw, d), jnp.float32),
          pltpu.SemaphoreType.DMA,
      ],
  )
  def k(rows_hbm, dst_hbm, out_hbm, idx_v, rows_v, sem):
    wid = lax.axis_index("s") * 2 + lax.axis_index("c")
    pltpu.sync_copy(dst_hbm.at[wid], idx_v)
    pltpu.sync_copy(rows_hbm.at[pl.ds(wid * bpw, bpw)], rows_v)
    pltpu.async_copy(rows_v, out_hbm.at[idx_v], sem).wait()

  return k(rows, dst2)


def _k_addpe(emb, pe):
  def body(e_ref, p_ref, o_ref):
    o_ref[...] = e_ref[...] + p_ref[...]

  return pl.pallas_call(
      body,
      grid=(NT,),
      in_specs=[pl.BlockSpec((BT, D), lambda i: (i, 0)),
                pl.BlockSpec((BT, D), lambda i: (i % (S // BT), 0))],
      out_specs=pl.BlockSpec((BT, D), lambda i: (i, 0)),
      out_shape=jax.ShapeDtypeStruct((T, D), jnp.float32),
  )(emb, pe)


def _k_attn_fused(x, g, b, w, wb):
  scale = 1.0 / float(np.sqrt(DH))
  nh2 = H // 2
  wb2 = wb.reshape(1, 3 * D)

  def body(x_ref, g_ref, b_ref, wq_ref, wk_ref, wv_ref, bq_ref, bk_ref,
           bv_ref, o_ref):
    z = _ln(x_ref[...], g_ref[...], b_ref[...])
    q = _dot_nt(z, wq_ref[...]) + bq_ref[...]
    kk = _dot_nt(z, wk_ref[...]) + bk_ref[...]
    v = _dot_nt(z, wv_ref[...]) + bv_ref[...]
    outs = []
    for u in range(2):
      qu = q[:, u * DH:(u + 1) * DH]
      ku = kk[:, u * DH:(u + 1) * DH]
      vu = v[:, u * DH:(u + 1) * DH]
      s_mat = _dot_nt(qu, ku) * scale
      m = jnp.max(s_mat, axis=1, keepdims=True)
      p = jnp.exp(s_mat - m)
      l = jnp.sum(p, axis=1, keepdims=True)
      outs.append(_dot_nn(p / l, vu))
    o_ref[...] = jnp.concatenate(outs, axis=1)

  return pl.pallas_call(
      body,
      grid=(B, nh2),
      in_specs=[
          pl.BlockSpec((S, D), lambda b, h: (b, 0)),
          pl.BlockSpec((1, D), lambda b, h: (0, 0)),
          pl.BlockSpec((1, D), lambda b, h: (0, 0)),
          pl.BlockSpec((2 * DH, D), lambda b, h: (h, 0)),
          pl.BlockSpec((2 * DH, D), lambda b, h: (nh2 + h, 0)),
          pl.BlockSpec((2 * DH, D), lambda b, h: (2 * nh2 + h, 0)),
          pl.BlockSpec((1, 2 * DH), lambda b, h: (0, h)),
          pl.BlockSpec((1, 2 * DH), lambda b, h: (0, nh2 + h)),
          pl.BlockSpec((1, 2 * DH), lambda b, h: (0, 2 * nh2 + h)),
      ],
      out_specs=pl.BlockSpec((S, 2 * DH), lambda b, h: (b, h)),
      out_shape=jax.ShapeDtypeStruct((T, D), jnp.float32),
  )(x, g.reshape(1, D), b.reshape(1, D), w, w, w, wb2, wb2, wb2)


def _k_proj_ln2_router(x, ao, w, wb, g, b, sw, sb):
  def body(x_ref, a_ref, w_ref, wb_ref, g_ref, b_ref, sw_ref, sb_ref,
           x1_ref, z_ref, rp_ref, rpm_ref, rt_ref):
    x1 = x_ref[...] + _dot_nt(a_ref[...], w_ref[...]) + wb_ref[...]
    x1_ref[...] = x1
    z = _ln(x1, g_ref[...], b_ref[...])
    z_ref[...] = z
    lg = _dot_nn(z, sw_ref[...]) + sb_ref[...]
    mx = jnp.max(lg, axis=1, keepdims=True)
    ex = jnp.exp(lg - mx)
    rp = ex / jnp.sum(ex, axis=1, keepdims=True)
    rp_ref[...] = rp
    pm = jnp.max(rp, axis=1, keepdims=True)
    rpm_ref[...] = pm
    ii = lax.broadcasted_iota(jnp.int32, (BT, E), 1)
    rt_ref[...] = jnp.min(jnp.where(rp == pm, ii, E), axis=1, keepdims=True)

  return pl.pallas_call(
      body,
      grid=(NT,),
      in_specs=[
          pl.BlockSpec((BT, D), lambda t: (t, 0)),
          pl.BlockSpec((BT, D), lambda t: (t, 0)),
          pl.BlockSpec((D, D), lambda t: (0, 0)),
          pl.BlockSpec((1, D), lambda t: (0, 0)),
          pl.BlockSpec((1, D), lambda t: (0, 0)),
          pl.BlockSpec((1, D), lambda t: (0, 0)),
          pl.BlockSpec((D, E), lambda t: (0, 0)),
          pl.BlockSpec((1, E), lambda t: (0, 0)),
      ],
      out_specs=[
          pl.BlockSpec((BT, D), lambda t: (t, 0)),
          pl.BlockSpec((BT, D), lambda t: (t, 0)),
          pl.BlockSpec((BT, E), lambda t: (t, 0)),
          pl.BlockSpec((BT, 1), lambda t: (t, 0)),
          pl.BlockSpec((BT, 1), lambda t: (t, 0)),
      ],
      out_shape=[
          jax.ShapeDtypeStruct((T, D), jnp.float32),
          jax.ShapeDtypeStruct((T, D), jnp.float32),
          jax.ShapeDtypeStruct((T, E), jnp.float32),
          jax.ShapeDtypeStruct((T, 1), jnp.float32),
          jax.ShapeDtypeStruct((T, 1), jnp.int32),
      ],
  )(x, ao, w, wb.reshape(1, D), g.reshape(1, D), b.reshape(1, D), sw,
    sb.reshape(1, E))


def _k_route(rt_s, rp_s):
  def body(rt_ref, rp_ref, dst_ref, gi_ref, kp_ref, cnt_ref, ps_ref, nd_ref):
    rt = rt_ref[...]
    oh = (rt == lax.broadcasted_iota(jnp.int32, (T, E), 1)).astype(jnp.int32)
    c = oh
    sh = 1
    while sh < T:
      c = c + jnp.concatenate(
          [jnp.zeros((sh, E), jnp.int32), c[:T - sh]], axis=0)
      sh *= 2
    pos = jnp.sum(c * oh, axis=1, keepdims=True) - 1
    keep = (pos < CAP).astype(jnp.int32)
    cnt_ref[...] = c[T - 1:T, :].astype(jnp.float32)
    ps_ref[...] = jnp.sum(rp_ref[...], axis=0, keepdims=True)
    nd_ref[...] = jnp.sum(1 - keep, axis=0, keepdims=True)
    dst_ref[...] = rt * CPAD + jnp.minimum(pos, CAP)
    gi_ref[...] = rt * CPAD + jnp.minimum(pos, CAP - 1)
    kp_ref[...] = keep

  return pl.pallas_call(
      body,
      out_shape=[
          jax.ShapeDtypeStruct((T, 1), jnp.int32),
          jax.ShapeDtypeStruct((T, 1), jnp.int32),
          jax.ShapeDtypeStruct((T, 1), jnp.int32),
          jax.ShapeDtypeStruct((1, E), jnp.float32),
          jax.ShapeDtypeStruct((1, E), jnp.float32),
          jax.ShapeDtypeStruct((1, 1), jnp.int32),
      ],
  )(rt_s, rp_s)


def _k_expert(buf, w1, b1, w2, b2, fast):
  def body(buf_ref, w1_ref, b1_ref, w2_ref, b2_ref, o_ref):
    j = pl.program_id(1)
    if fast:
      bb = buf_ref[...].astype(jnp.bfloat16)
      h = jnp.maximum(
          _dot_nn(bb, w1_ref[0].astype(jnp.bfloat16),
                  precision=lax.Precision.DEFAULT) + b1_ref[0], 0.0)
      part = _dot_nn(h.astype(jnp.bfloat16), w2_ref[0].astype(jnp.bfloat16),
                     precision=lax.Precision.DEFAULT)
    else:
      h = jnp.maximum(_dot_nn(buf_ref[...], w1_ref[0]) + b1_ref[0], 0.0)
      part = _dot_nn(h, w2_ref[0])

    @pl.when(j == 0)
    def _():
      o_ref[...] = part + b2_ref[0]

    @pl.when(j != 0)
    def _():
      o_ref[...] = o_ref[...] + part

  return pl.pallas_call(
      body,
      grid=(E, DFF // DC),
      in_specs=[
          pl.BlockSpec((CPAD, D), lambda e, j: (e, 0)),
          pl.BlockSpec((1, D, DC), lambda e, j: (e, 0, j)),
          pl.BlockSpec((1, 1, DC), lambda e, j: (e, 0, j)),
          pl.BlockSpec((1, DC, D), lambda e, j: (e, j, 0)),
          pl.BlockSpec((1, 1, D), lambda e, j: (e, 0, 0)),
      ],
      out_specs=pl.BlockSpec((CPAD, D), lambda e, j: (e, 0)),
      out_shape=jax.ShapeDtypeStruct((E * CPAD, D), jnp.float32),
  )(buf, w1, b1.reshape(E, 1, DFF), w2, b2.reshape(E, 1, D))


def _k_combine(x1, z, gat, kp, rpm):
  def body(x_ref, z_ref, g_ref, k_ref, p_ref, o_ref):
    sel = jnp.where(k_ref[...] != 0, g_ref[...], z_ref[...])
    o_ref[...] = x_ref[...] + sel * p_ref[...]

  return pl.pallas_call(
      body,
      grid=(NT,),
      in_specs=[
          pl.BlockSpec((BT, D), lambda t: (t, 0)),
          pl.BlockSpec((BT, D), lambda t: (t, 0)),
          pl.BlockSpec((BT, D), lambda t: (t, 0)),
          pl.BlockSpec((BT, 1), lambda t: (t, 0)),
          pl.BlockSpec((BT, 1), lambda t: (t, 0)),
      ],
      out_specs=pl.BlockSpec((BT, D), lambda t: (t, 0)),
      out_shape=jax.ShapeDtypeStruct((T, D), jnp.float32),
  )(x1, z, gat, kp, rpm)


def _k_combine_fln(x1, z, gat, kp, rpm, g, b):
  def body(x_ref, z_ref, g2_ref, k_ref, p_ref, g_ref, b_ref, o_ref):
    sel = jnp.where(k_ref[...] != 0, g2_ref[...], z_ref[...])
    xf = x_ref[...] + sel * p_ref[...]
    o_ref[...] = _ln(xf, g_ref[...], b_ref[...]).astype(jnp.bfloat16)

  return pl.pallas_call(
      body,
      grid=(NT,),
      in_specs=[
          pl.BlockSpec((BT, D), lambda t: (t, 0)),
          pl.BlockSpec((BT, D), lambda t: (t, 0)),
          pl.BlockSpec((BT, D), lambda t: (t, 0)),
          pl.BlockSpec((BT, 1), lambda t: (t, 0)),
          pl.BlockSpec((BT, 1), lambda t: (t, 0)),
          pl.BlockSpec((1, D), lambda t: (0, 0)),
          pl.BlockSpec((1, D), lambda t: (0, 0)),
      ],
      out_specs=pl.BlockSpec((BT, D), lambda t: (t, 0)),
      out_shape=jax.ShapeDtypeStruct((T, D), jnp.bfloat16),
  )(x1, z, gat, kp, rpm, g.reshape(1, D), b.reshape(1, D))


def _k_dec_lse(xb, w, db):
  def body(x_ref, w_ref, db_ref, lse_ref, m_acc, s_acc):
    v = pl.program_id(0)

    @pl.when(v == 0)
    def _():
      m_acc[...] = jnp.full((T, 1), -1e30, jnp.float32)
      s_acc[...] = jnp.zeros((T, 1), jnp.float32)

    wb = w_ref[...].astype(jnp.bfloat16)
    for c in range(NT):
      xc = x_ref[pl.ds(c * BT, BT), :]
      lg = lax.dot_general(xc, wb, (((1,), (1,)), ((), ())),
                           preferred_element_type=jnp.float32) + db_ref[...]
      mo = m_acc[pl.ds(c * BT, BT), :]
      so = s_acc[pl.ds(c * BT, BT), :]
      tm = jnp.max(lg, axis=1, keepdims=True)
      mn = jnp.maximum(mo, tm)
      sn = so * jnp.exp(mo - mn) + jnp.sum(jnp.exp(lg - mn), axis=1,
                                           keepdims=True)
      m_acc[pl.ds(c * BT, BT), :] = mn
      s_acc[pl.ds(c * BT, BT), :] = sn

    @pl.when(v == NV - 1)
    def _():
      lse_ref[...] = m_acc[...] + jnp.log(s_acc[...])

  return pl.pallas_call(
      body,
      grid=(NV,),
      in_specs=[
          pl.BlockSpec((T, D), lambda v: (0, 0)),
          pl.BlockSpec((VT, D), lambda v: (v, 0)),
          pl.BlockSpec((1, VT), lambda v: (0, v)),
      ],
      out_specs=pl.BlockSpec((T, 1), lambda v: (0, 0)),
      out_shape=jax.ShapeDtypeStruct((T, 1), jnp.float32),
      scratch_shapes=[pltpu.VMEM((T, 1), jnp.float32),
                      pltpu.VMEM((T, 1), jnp.float32)],
  )(xb, w, db)


def _k_dec_out(xb, w, db, lse):
  def body(x_ref, w_ref, db_ref, l_ref, o_ref, wb_s):
    t = pl.program_id(1)

    @pl.when(t == 0)
    def _():
      wb_s[...] = w_ref[...].astype(jnp.bfloat16)

    xc = x_ref[pl.ds(t * BT, BT), :]
    lg = lax.dot_general(xc, wb_s[...], (((1,), (1,)), ((), ())),
                         preferred_element_type=jnp.float32) + db_ref[...]
    o_ref[...] = lg - l_ref[pl.ds(t * BT, BT), :]

  return pl.pallas_call(
      body,
      grid=(NV, NT),
      in_specs=[
          pl.BlockSpec((T, D), lambda v, t: (0, 0)),
          pl.BlockSpec((VT, D), lambda v, t: (v, 0)),
          pl.BlockSpec((1, VT), lambda v, t: (0, v)),
          pl.BlockSpec((T, 1), lambda v, t: (0, 0)),
      ],
      out_specs=pl.BlockSpec((BT, VT), lambda v, t: (t, v)),
      out_shape=jax.ShapeDtypeStruct((T, VOCAB), jnp.float32),
      scratch_shapes=[pltpu.VMEM((VT, D), jnp.bfloat16)],
  )(xb, w, db, lse)


def _to_s(a):
  return a.reshape(B, S, -1).transpose(1, 0, 2).reshape(T, -1)


def _to_b(a):
  return a.reshape(S, B, -1).transpose(1, 0, 2).reshape(T, -1)


def kernel(input_chars, embed_w, ln1_g, ln1_b, ln2_g, ln2_b, attn_in_w,
           attn_in_b, attn_out_w, attn_out_b, switch_w, switch_b, exp_w1,
           exp_b1, exp_w2, exp_b2, fln_g, fln_b, dec_w, dec_b):
  idx = input_chars.reshape(T)
  emb = _sc_gather_rows(embed_w, idx)
  x = _k_addpe(emb, jnp.asarray(_pe_np()))
  counts_l, ps_l, nd_l, rpm_l = [], [], [], []
  xb = None
  for i in range(2):
    ao = _k_attn_fused(x, ln1_g[i], ln1_b[i], attn_in_w[i], attn_in_b[i])
    x1, z, rp, rpm, rt = _k_proj_ln2_router(
        x, ao, attn_out_w[i], attn_out_b[i], ln2_g[i], ln2_b[i], switch_w[i],
        switch_b[i])
    dst_s, gi_s, kp_s, cnt, ps, nd = _k_route(_to_s(rt), _to_s(rp))
    dst = _to_b(dst_s).reshape(T)
    gi = _to_b(gi_s).reshape(T)
    kp = _to_b(kp_s)
    buf = _sc_scatter_rows(z, dst, E * CPAD)
    eo = _k_expert(buf, exp_w1[i], exp_b1[i], exp_w2[i], exp_b2[i],
                   fast=(i == 1))
    gat = _sc_gather_rows(eo, gi)
    if i == 0:
      x = _k_combine(x1, z, gat, kp, rpm)
    else:
      xb = _k_combine_fln(x1, z, gat, kp, rpm, fln_g, fln_b)
    counts_l.append(cnt.reshape(E))
    ps_l.append(ps.reshape(E))
    nd_l.append(nd.reshape(()))
    rpm_l.append(_to_s(rpm).reshape(T))
  db2 = dec_b.reshape(1, VOCAB)
  lse = _k_dec_lse(xb, dec_w, db2)
  lp = _k_dec_out(xb, dec_w, db2, lse)
  return (lp.reshape(B, S, VOCAB), jnp.stack(counts_l), jnp.stack(ps_l),
          jnp.stack(nd_l), jnp.stack(rpm_l))


# speedup vs baseline: 1.3155x; 1.0067x over previous
import functools

import numpy as np
import jax
import jax.numpy as jnp
from jax import lax
from jax.experimental import pallas as pl
from jax.experimental.pallas import tpu as pltpu
from jax.experimental.pallas import tpu_sc as plsc

B = 2
S = 1024
D = 1024
H = 16
DH = D // H
E = 16
DFF = 2048
T = B * S
CAP = int(1.2 * T / E)
CPAD = 160
VOCAB = 32000
BT = 256
NT = T // BT
VT = 3200
NV = VOCAB // VT
DC = 512
_NW = 32

def _split_hl(a):
  ah = a.astype(jnp.bfloat16)
  al = (a - ah.astype(jnp.float32)).astype(jnp.bfloat16)
  return ah, al


def _dot_nt(a, b, precision=None):
  dims = (((1,), (1,)), ((), ()))
  if precision is not None:
    return lax.dot_general(a, b, dims, precision=precision,
                           preferred_element_type=jnp.float32)
  ah, al = _split_hl(a)
  bh, bl = _split_hl(b)
  d = lambda x, y: lax.dot_general(x, y, dims,
                                   preferred_element_type=jnp.float32)
  return d(ah, bh) + (d(ah, bl) + d(al, bh))


def _dot_nn(a, b, precision=None):
  dims = (((1,), (0,)), ((), ()))
  if precision is not None:
    return lax.dot_general(a, b, dims, precision=precision,
                           preferred_element_type=jnp.float32)
  ah, al = _split_hl(a)
  bh, bl = _split_hl(b)
  d = lambda x, y: lax.dot_general(x, y, dims,
                                   preferred_element_type=jnp.float32)
  return d(ah, bh) + (d(ah, bl) + d(al, bh))


def _ln(x, g, b):
  m = jnp.mean(x, axis=-1, keepdims=True)
  v = jnp.mean((x - m) ** 2, axis=-1, keepdims=True)
  return (x - m) / jnp.sqrt(v + 1e-5) * g + b


def _pe_np():
  pos = np.arange(S, dtype=np.float32)[:, None]
  div = np.exp(np.arange(0, D, 2, dtype=np.float32) * (-np.log(10000.0) / D))
  pe = np.zeros((S, D), dtype=np.float32)
  pe[:, 0::2] = np.sin(pos * div)
  pe[:, 1::2] = np.cos(pos * div)
  return pe


def _sc_gather_rows(table, idx):
  t = idx.shape[0]
  d = table.shape[1]
  bpw = t // _NW
  idx2 = idx.reshape(_NW, bpw)
  mesh = plsc.VectorSubcoreMesh(core_axis_name="c", subcore_axis_name="s")

  @functools.partial(
      pl.kernel, mesh=mesh,
      out_type=jax.ShapeDtypeStruct((t, d), jnp.float32),
      scratch_types=[
          pltpu.VMEM((bpw,), jnp.int32),
          pltpu.VMEM((bpw, d), jnp.float32),
          pltpu.SemaphoreType.DMA,
      ],
  )
  def k(table_hbm, idx_hbm, out_hbm, idx_v, rows_v, sem):
    wid = lax.axis_index("s") * 2 + lax.axis_index("c")
    pltpu.sync_copy(idx_hbm.at[wid], idx_v)
    pltpu.async_copy(table_hbm.at[idx_v], rows_v, sem).wait()
    pltpu.sync_copy(rows_v, out_hbm.at[pl.ds(wid * bpw, bpw)])

  return k(table, idx2)


def _k_addpe(emb, pe):
  def body(e_ref, p_ref, o_ref):
    o_ref[...] = e_ref[...] + p_ref[...]

  return pl.pallas_call(
      body,
      grid=(NT,),
      in_specs=[pl.BlockSpec((BT, D), lambda i: (i, 0)),
                pl.BlockSpec((BT, D), lambda i: (i % (S // BT), 0))],
      out_specs=pl.BlockSpec((BT, D), lambda i: (i, 0)),
      out_shape=jax.ShapeDtypeStruct((T, D), jnp.float32),
  )(emb, pe)


def _k_attn_fused(x, g, b, w, wb):
  scale = 1.0 / float(np.sqrt(DH))
  nh2 = H // 2
  wb2 = wb.reshape(1, 3 * D)

  def body(x_ref, g_ref, b_ref, wq_ref, wk_ref, wv_ref, bq_ref, bk_ref,
           bv_ref, o_ref):
    z = _ln(x_ref[...], g_ref[...], b_ref[...])
    q = _dot_nt(z, wq_ref[...]) + bq_ref[...]
    kk = _dot_nt(z, wk_ref[...]) + bk_ref[...]
    v = _dot_nt(z, wv_ref[...]) + bv_ref[...]
    outs = []
    for u in range(2):
      qu = q[:, u * DH:(u + 1) * DH]
      ku = kk[:, u * DH:(u + 1) * DH]
      vu = v[:, u * DH:(u + 1) * DH]
      s_mat = _dot_nt(qu, ku) * scale
      m = jnp.max(s_mat, axis=1, keepdims=True)
      p = jnp.exp(s_mat - m)
      l = jnp.sum(p, axis=1, keepdims=True)
      outs.append(_dot_nn(p / l, vu))
    o_ref[...] = jnp.concatenate(outs, axis=1)

  return pl.pallas_call(
      body,
      grid=(B, nh2),
      in_specs=[
          pl.BlockSpec((S, D), lambda b, h: (b, 0)),
          pl.BlockSpec((1, D), lambda b, h: (0, 0)),
          pl.BlockSpec((1, D), lambda b, h: (0, 0)),
          pl.BlockSpec((2 * DH, D), lambda b, h: (h, 0)),
          pl.BlockSpec((2 * DH, D), lambda b, h: (nh2 + h, 0)),
          pl.BlockSpec((2 * DH, D), lambda b, h: (2 * nh2 + h, 0)),
          pl.BlockSpec((1, 2 * DH), lambda b, h: (0, h)),
          pl.BlockSpec((1, 2 * DH), lambda b, h: (0, nh2 + h)),
          pl.BlockSpec((1, 2 * DH), lambda b, h: (0, 2 * nh2 + h)),
      ],
      out_specs=pl.BlockSpec((S, 2 * DH), lambda b, h: (b, h)),
      out_shape=jax.ShapeDtypeStruct((T, D), jnp.float32),
  )(x, g.reshape(1, D), b.reshape(1, D), w, w, w, wb2, wb2, wb2)


def _k_proj_ln2_router(x, ao, w, wb, g, b, sw, sb):
  def body(x_ref, a_ref, w_ref, wb_ref, g_ref, b_ref, sw_ref, sb_ref,
           x1_ref, z_ref, zh_ref, zm_ref, zl_ref, rp_ref, rpm_ref, rt_ref):
    x1 = x_ref[...] + _dot_nt(a_ref[...], w_ref[...]) + wb_ref[...]
    x1_ref[...] = x1
    z = _ln(x1, g_ref[...], b_ref[...])
    z_ref[...] = z
    zh = z.astype(jnp.bfloat16)
    zh_ref[...] = zh
    r1 = z - zh.astype(jnp.float32)
    zm = r1.astype(jnp.bfloat16)
    zm_ref[...] = zm
    zl_ref[...] = (r1 - zm.astype(jnp.float32)).astype(jnp.bfloat16)
    lg = _dot_nn(z, sw_ref[...]) + sb_ref[...]
    mx = jnp.max(lg, axis=1, keepdims=True)
    ex = jnp.exp(lg - mx)
    rp = ex / jnp.sum(ex, axis=1, keepdims=True)
    rp_ref[...] = rp
    pm = jnp.max(rp, axis=1, keepdims=True)
    rpm_ref[...] = pm
    ii = lax.broadcasted_iota(jnp.int32, (BT, E), 1)
    rt_ref[...] = jnp.min(jnp.where(rp == pm, ii, E), axis=1, keepdims=True)

  return pl.pallas_call(
      body,
      grid=(NT,),
      in_specs=[
          pl.BlockSpec((BT, D), lambda t: (t, 0)),
          pl.BlockSpec((BT, D), lambda t: (t, 0)),
          pl.BlockSpec((D, D), lambda t: (0, 0)),
          pl.BlockSpec((1, D), lambda t: (0, 0)),
          pl.BlockSpec((1, D), lambda t: (0, 0)),
          pl.BlockSpec((1, D), lambda t: (0, 0)),
          pl.BlockSpec((D, E), lambda t: (0, 0)),
          pl.BlockSpec((1, E), lambda t: (0, 0)),
      ],
      out_specs=[
          pl.BlockSpec((BT, D), lambda t: (t, 0)),
          pl.BlockSpec((BT, D), lambda t: (t, 0)),
          pl.BlockSpec((BT, D), lambda t: (t, 0)),
          pl.BlockSpec((BT, D), lambda t: (t, 0)),
          pl.BlockSpec((BT, D), lambda t: (t, 0)),
          pl.BlockSpec((BT, E), lambda t: (t, 0)),
          pl.BlockSpec((BT, 1), lambda t: (t, 0)),
          pl.BlockSpec((BT, 1), lambda t: (t, 0)),
      ],
      out_shape=[
          jax.ShapeDtypeStruct((T, D), jnp.float32),
          jax.ShapeDtypeStruct((T, D), jnp.float32),
          jax.ShapeDtypeStruct((T, D), jnp.bfloat16),
          jax.ShapeDtypeStruct((T, D), jnp.bfloat16),
          jax.ShapeDtypeStruct((T, D), jnp.bfloat16),
          jax.ShapeDtypeStruct((T, E), jnp.float32),
          jax.ShapeDtypeStruct((T, 1), jnp.float32),
          jax.ShapeDtypeStruct((T, 1), jnp.int32),
      ],
  )(x, ao, w, wb.reshape(1, D), g.reshape(1, D), b.reshape(1, D), sw,
    sb.reshape(1, E))


def _k_route(rt_s, rp_s):
  def body(rt_ref, rp_ref, dst_ref, gi_ref, kp_ref, cnt_ref, ps_ref, nd_ref):
    rt = rt_ref[...]
    oh = (rt == lax.broadcasted_iota(jnp.int32, (T, E), 1)).astype(jnp.int32)
    c = oh
    sh = 1
    while sh < T:
      c = c + jnp.concatenate(
          [jnp.zeros((sh, E), jnp.int32), c[:T - sh]], axis=0)
      sh *= 2
    pos = jnp.sum(c * oh, axis=1, keepdims=True) - 1
    keep = (pos < CAP).astype(jnp.int32)
    cnt_ref[...] = c[T - 1:T, :].astype(jnp.float32)
    ps_ref[...] = jnp.sum(rp_ref[...], axis=0, keepdims=True)
    nd_ref[...] = jnp.sum(1 - keep, axis=0, keepdims=True)
    dst_ref[...] = rt * CPAD + jnp.minimum(pos, CAP)
    gi_ref[...] = rt * CPAD + jnp.minimum(pos, CAP - 1)
    kp_ref[...] = keep

  return pl.pallas_call(
      body,
      out_shape=[
          jax.ShapeDtypeStruct((T, 1), jnp.int32),
          jax.ShapeDtypeStruct((T, 1), jnp.int32),
          jax.ShapeDtypeStruct((T, 1), jnp.int32),
          jax.ShapeDtypeStruct((1, E), jnp.float32),
          jax.ShapeDtypeStruct((1, E), jnp.float32),
          jax.ShapeDtypeStruct((1, 1), jnp.int32),
      ],
  )(rt_s, rp_s)


def _k_expert(zh, zm, zl, dst, w1, b1, w2, b2, fast):
  nj = DFF // DC
  ddef = lax.Precision.DEFAULT

  def body(zh_ref, zm_ref, zl_ref, dst_ref, w1_ref, b1_ref, w2_ref, b2_ref,
           *out_and_scratch):
    if fast:
      eh_ref, be_s, acc_s = out_and_scratch
      em_ref = el_ref = None
    else:
      eh_ref, em_ref, el_ref, be_s, acc_s = out_and_scratch
    e = pl.program_id(0)
    j = pl.program_id(1)

    @pl.when(j == 0)
    def _():
      slots = e * CPAD + lax.broadcasted_iota(jnp.int32, (CPAD, T), 0)
      p = (slots == dst_ref[...]).astype(jnp.bfloat16)
      be = _dot_nn(p, zh_ref[...], precision=ddef)
      if not fast:
        be = be + (_dot_nn(p, zm_ref[...], precision=ddef) +
                   _dot_nn(p, zl_ref[...], precision=ddef))
      be_s[...] = be

    if fast:
      bb = be_s[...].astype(jnp.bfloat16)
      h = jnp.maximum(
          _dot_nn(bb, w1_ref[0].astype(jnp.bfloat16), precision=ddef)
          + b1_ref[0], 0.0)
      part = _dot_nn(h.astype(jnp.bfloat16), w2_ref[0].astype(jnp.bfloat16),
                     precision=ddef)
    else:
      h = jnp.maximum(_dot_nn(be_s[...], w1_ref[0]) + b1_ref[0], 0.0)
      part = _dot_nn(h, w2_ref[0])

    @pl.when(j == 0)
    def _():
      acc_s[...] = part + b2_ref[0]

    @pl.when(j != 0)
    def _():
      acc_s[...] = acc_s[...] + part

    @pl.when(j == nj - 1)
    def _():
      eo = acc_s[...]
      eh = eo.astype(jnp.bfloat16)
      eh_ref[...] = eh
      if not fast:
        r1 = eo - eh.astype(jnp.float32)
        em = r1.astype(jnp.bfloat16)
        em_ref[...] = em
        el_ref[...] = (r1 - em.astype(jnp.float32)).astype(jnp.bfloat16)

  eo_shape = jax.ShapeDtypeStruct((E * CPAD, D), jnp.bfloat16)
  n_out = 1 if fast else 3
  out = pl.pallas_call(
      body,
      grid=(E, nj),
      in_specs=[
          pl.BlockSpec((T, D), lambda e, j: (0, 0)),
          pl.BlockSpec((T, D), lambda e, j: (0, 0)),
          pl.BlockSpec((T, D), lambda e, j: (0, 0)),
          pl.BlockSpec((1, T), lambda e, j: (0, 0)),
          pl.BlockSpec((1, D, DC), lambda e, j: (e, 0, j)),
          pl.BlockSpec((1, 1, DC), lambda e, j: (e, 0, j)),
          pl.BlockSpec((1, DC, D), lambda e, j: (e, j, 0)),
          pl.BlockSpec((1, 1, D), lambda e, j: (e, 0, 0)),
      ],
      out_specs=[pl.BlockSpec((CPAD, D), lambda e, j: (e, 0))] * n_out,
      out_shape=[eo_shape] * n_out,
      scratch_shapes=[pltpu.VMEM((CPAD, D), jnp.float32),
                      pltpu.VMEM((CPAD, D), jnp.float32)],
  )(zh, zm, zl, dst, w1, b1.reshape(E, 1, DFF), w2, b2.reshape(E, 1, D))
  return (out[0], None, None) if fast else tuple(out)


def _gather_eo(gi_blk, eh_ref, em_ref, el_ref):
  ddef = lax.Precision.DEFAULT
  slots = lax.broadcasted_iota(jnp.int32, (BT, E * CPAD), 1)
  g_1h = (slots == gi_blk).astype(jnp.bfloat16)
  gat = _dot_nn(g_1h, eh_ref[...], precision=ddef)
  if em_ref is not None:
    gat = gat + (_dot_nn(g_1h, em_ref[...], precision=ddef) +
                 _dot_nn(g_1h, el_ref[...], precision=ddef))
  return gat


def _k_combine(x1, z, eh, em, el, gi, kp, rpm):
  def body(x_ref, z_ref, eh_ref, em_ref, el_ref, gi_ref, k_ref, p_ref, o_ref):
    gat = _gather_eo(gi_ref[...], eh_ref, em_ref, el_ref)
    sel = jnp.where(k_ref[...] != 0, gat, z_ref[...])
    o_ref[...] = x_ref[...] + sel * p_ref[...]

  return pl.pallas_call(
      body,
      grid=(NT,),
      in_specs=[
          pl.BlockSpec((BT, D), lambda t: (t, 0)),
          pl.BlockSpec((BT, D), lambda t: (t, 0)),
          pl.BlockSpec((E * CPAD, D), lambda t: (0, 0)),
          pl.BlockSpec((E * CPAD, D), lambda t: (0, 0)),
          pl.BlockSpec((E * CPAD, D), lambda t: (0, 0)),
          pl.BlockSpec((BT, 1), lambda t: (t, 0)),
          pl.BlockSpec((BT, 1), lambda t: (t, 0)),
          pl.BlockSpec((BT, 1), lambda t: (t, 0)),
      ],
      out_specs=pl.BlockSpec((BT, D), lambda t: (t, 0)),
      out_shape=jax.ShapeDtypeStruct((T, D), jnp.float32),
  )(x1, z, eh, em, el, gi, kp, rpm)


def _k_combine_fln(x1, z, eh, gi, kp, rpm, g, b):
  def body(x_ref, z_ref, eh_ref, gi_ref, k_ref, p_ref, g_ref, b_ref, o_ref):
    gat = _gather_eo(gi_ref[...], eh_ref, None, None)
    sel = jnp.where(k_ref[...] != 0, gat, z_ref[...])
    xf = x_ref[...] + sel * p_ref[...]
    o_ref[...] = _ln(xf, g_ref[...], b_ref[...]).astype(jnp.bfloat16)

  return pl.pallas_call(
      body,
      grid=(NT,),
      in_specs=[
          pl.BlockSpec((BT, D), lambda t: (t, 0)),
          pl.BlockSpec((BT, D), lambda t: (t, 0)),
          pl.BlockSpec((E * CPAD, D), lambda t: (0, 0)),
          pl.BlockSpec((BT, 1), lambda t: (t, 0)),
          pl.BlockSpec((BT, 1), lambda t: (t, 0)),
          pl.BlockSpec((BT, 1), lambda t: (t, 0)),
          pl.BlockSpec((1, D), lambda t: (0, 0)),
          pl.BlockSpec((1, D), lambda t: (0, 0)),
      ],
      out_specs=pl.BlockSpec((BT, D), lambda t: (t, 0)),
      out_shape=jax.ShapeDtypeStruct((T, D), jnp.bfloat16),
  )(x1, z, eh, gi, kp, rpm, g.reshape(1, D), b.reshape(1, D))


def _k_dec_lse(xb, w, db):
  def body(x_ref, w_ref, db_ref, lse_ref, m_acc, s_acc):
    v = pl.program_id(0)

    @pl.when(v == 0)
    def _():
      m_acc[...] = jnp.full((T, 1), -1e30, jnp.float32)
      s_acc[...] = jnp.zeros((T, 1), jnp.float32)

    wb = w_ref[...].astype(jnp.bfloat16)
    for c in range(NT):
      xc = x_ref[pl.ds(c * BT, BT), :]
      lg = lax.dot_general(xc, wb, (((1,), (1,)), ((), ())),
                           preferred_element_type=jnp.float32) + db_ref[...]
      mo = m_acc[pl.ds(c * BT, BT), :]
      so = s_acc[pl.ds(c * BT, BT), :]
      tm = jnp.max(lg, axis=1, keepdims=True)
      mn = jnp.maximum(mo, tm)
      sn = so * jnp.exp(mo - mn) + jnp.sum(jnp.exp(lg - mn), axis=1,
                                           keepdims=True)
      m_acc[pl.ds(c * BT, BT), :] = mn
      s_acc[pl.ds(c * BT, BT), :] = sn

    @pl.when(v == NV - 1)
    def _():
      lse_ref[...] = m_acc[...] + jnp.log(s_acc[...])

  return pl.pallas_call(
      body,
      grid=(NV,),
      in_specs=[
          pl.BlockSpec((T, D), lambda v: (0, 0)),
          pl.BlockSpec((VT, D), lambda v: (v, 0)),
          pl.BlockSpec((1, VT), lambda v: (0, v)),
      ],
      out_specs=pl.BlockSpec((T, 1), lambda v: (0, 0)),
      out_shape=jax.ShapeDtypeStruct((T, 1), jnp.float32),
      scratch_shapes=[pltpu.VMEM((T, 1), jnp.float32),
                      pltpu.VMEM((T, 1), jnp.float32)],
  )(xb, w, db)


def _k_dec_out(xb, w, db, lse):
  def body(x_ref, w_ref, db_ref, l_ref, o_ref, wb_s):
    t = pl.program_id(1)

    @pl.when(t == 0)
    def _():
      wb_s[...] = w_ref[...].astype(jnp.bfloat16)

    xc = x_ref[pl.ds(t * BT, BT), :]
    lg = lax.dot_general(xc, wb_s[...], (((1,), (1,)), ((), ())),
                         preferred_element_type=jnp.float32) + db_ref[...]
    o_ref[...] = lg - l_ref[pl.ds(t * BT, BT), :]

  return pl.pallas_call(
      body,
      grid=(NV, NT),
      in_specs=[
          pl.BlockSpec((T, D), lambda v, t: (0, 0)),
          pl.BlockSpec((VT, D), lambda v, t: (v, 0)),
          pl.BlockSpec((1, VT), lambda v, t: (0, v)),
          pl.BlockSpec((T, 1), lambda v, t: (0, 0)),
      ],
      out_specs=pl.BlockSpec((BT, VT), lambda v, t: (t, v)),
      out_shape=jax.ShapeDtypeStruct((T, VOCAB), jnp.float32),
      scratch_shapes=[pltpu.VMEM((VT, D), jnp.bfloat16)],
  )(xb, w, db, lse)


def _to_s(a):
  return a.reshape(B, S, -1).transpose(1, 0, 2).reshape(T, -1)


def _to_b(a):
  return a.reshape(S, B, -1).transpose(1, 0, 2).reshape(T, -1)


def kernel(input_chars, embed_w, ln1_g, ln1_b, ln2_g, ln2_b, attn_in_w,
           attn_in_b, attn_out_w, attn_out_b, switch_w, switch_b, exp_w1,
           exp_b1, exp_w2, exp_b2, fln_g, fln_b, dec_w, dec_b):
  idx = input_chars.reshape(T)
  emb = _sc_gather_rows(embed_w, idx)
  x = _k_addpe(emb, jnp.asarray(_pe_np()))
  counts_l, ps_l, nd_l, rpm_l = [], [], [], []
  xb = None
  for i in range(2):
    fast = i == 1
    ao = _k_attn_fused(x, ln1_g[i], ln1_b[i], attn_in_w[i], attn_in_b[i])
    x1, z, zh, zm, zl, rp, rpm, rt = _k_proj_ln2_router(
        x, ao, attn_out_w[i], attn_out_b[i], ln2_g[i], ln2_b[i], switch_w[i],
        switch_b[i])
    dst_s, gi_s, kp_s, cnt, ps, nd = _k_route(_to_s(rt), _to_s(rp))
    dst = _to_b(dst_s).reshape(1, T)
    gi = _to_b(gi_s)
    kp = _to_b(kp_s)
    eh, em, el = _k_expert(zh, zm, zl, dst, exp_w1[i], exp_b1[i], exp_w2[i],
                           exp_b2[i], fast=fast)
    if not fast:
      x = _k_combine(x1, z, eh, em, el, gi, kp, rpm)
    else:
      xb = _k_combine_fln(x1, z, eh, gi, kp, rpm, fln_g, fln_b)
    counts_l.append(cnt.reshape(E))
    ps_l.append(ps.reshape(E))
    nd_l.append(nd.reshape(()))
    rpm_l.append(_to_s(rpm).reshape(T))
  db2 = dec_b.reshape(1, VOCAB)
  lse = _k_dec_lse(xb, dec_w, db2)
  lp = _k_dec_out(xb, dec_w, db2, lse)
  return (lp.reshape(B, S, VOCAB), jnp.stack(counts_l), jnp.stack(ps_l),
          jnp.stack(nd_l), jnp.stack(rpm_l))


# speedup vs baseline: 1.3424x; 1.0205x over previous
import functools

import numpy as np
import jax
import jax.numpy as jnp
from jax import lax
from jax.experimental import pallas as pl
from jax.experimental.pallas import tpu as pltpu
from jax.experimental.pallas import tpu_sc as plsc

B = 2
S = 1024
D = 1024
H = 16
DH = D // H
E = 16
DFF = 2048
T = B * S
CAP = int(1.2 * T / E)
CPAD = 160
VOCAB = 32000
BT = 256
NT = T // BT
VT = 3200
NV = VOCAB // VT
DC = 512
_NW = 32

def _split_hl(a):
  ah = a.astype(jnp.bfloat16)
  al = (a - ah.astype(jnp.float32)).astype(jnp.bfloat16)
  return ah, al


def _dot_nt(a, b, precision=None):
  dims = (((1,), (1,)), ((), ()))
  if precision is not None:
    return lax.dot_general(a, b, dims, precision=precision,
                           preferred_element_type=jnp.float32)
  ah, al = _split_hl(a)
  bh, bl = _split_hl(b)
  d = lambda x, y: lax.dot_general(x, y, dims,
                                   preferred_element_type=jnp.float32)
  return d(ah, bh) + (d(ah, bl) + d(al, bh))


def _dot_nn(a, b, precision=None):
  dims = (((1,), (0,)), ((), ()))
  if precision is not None:
    return lax.dot_general(a, b, dims, precision=precision,
                           preferred_element_type=jnp.float32)
  ah, al = _split_hl(a)
  bh, bl = _split_hl(b)
  d = lambda x, y: lax.dot_general(x, y, dims,
                                   preferred_element_type=jnp.float32)
  return d(ah, bh) + (d(ah, bl) + d(al, bh))


def _ln(x, g, b):
  m = jnp.mean(x, axis=-1, keepdims=True)
  v = jnp.mean((x - m) ** 2, axis=-1, keepdims=True)
  return (x - m) / jnp.sqrt(v + 1e-5) * g + b


def _pe_np():
  pos = np.arange(S, dtype=np.float32)[:, None]
  div = np.exp(np.arange(0, D, 2, dtype=np.float32) * (-np.log(10000.0) / D))
  pe = np.zeros((S, D), dtype=np.float32)
  pe[:, 0::2] = np.sin(pos * div)
  pe[:, 1::2] = np.cos(pos * div)
  return pe


def _sc_gather_rows(table, idx):
  t = idx.shape[0]
  d = table.shape[1]
  bpw = t // _NW
  idx2 = idx.reshape(_NW, bpw)
  mesh = plsc.VectorSubcoreMesh(core_axis_name="c", subcore_axis_name="s")

  @functools.partial(
      pl.kernel, mesh=mesh,
      out_type=jax.ShapeDtypeStruct((t, d), jnp.float32),
      scratch_types=[
          pltpu.VMEM((bpw,), jnp.int32),
          pltpu.VMEM((bpw, d), jnp.float32),
          pltpu.SemaphoreType.DMA,
      ],
  )
  def k(table_hbm, idx_hbm, out_hbm, idx_v, rows_v, sem):
    wid = lax.axis_index("s") * 2 + lax.axis_index("c")
    pltpu.sync_copy(idx_hbm.at[wid], idx_v)
    pltpu.async_copy(table_hbm.at[idx_v], rows_v, sem).wait()
    pltpu.sync_copy(rows_v, out_hbm.at[pl.ds(wid * bpw, bpw)])

  return k(table, idx2)


def _k_addpe_ln(emb, pe, g, b):
  def body(e_ref, p_ref, g_ref, b_ref, x_ref, z_ref):
    x = e_ref[...] + p_ref[...]
    x_ref[...] = x
    z_ref[...] = _ln(x, g_ref[...], b_ref[...])

  return pl.pallas_call(
      body,
      grid=(NT,),
      in_specs=[pl.BlockSpec((BT, D), lambda i: (i, 0)),
                pl.BlockSpec((BT, D), lambda i: (i % (S // BT), 0)),
                pl.BlockSpec((1, D), lambda i: (0, 0)),
                pl.BlockSpec((1, D), lambda i: (0, 0))],
      out_specs=[pl.BlockSpec((BT, D), lambda i: (i, 0)),
                 pl.BlockSpec((BT, D), lambda i: (i, 0))],
      out_shape=[jax.ShapeDtypeStruct((T, D), jnp.float32),
                 jax.ShapeDtypeStruct((T, D), jnp.float32)],
  )(emb, pe, g.reshape(1, D), b.reshape(1, D))


def _k_attn_fused(z, w, wb):
  scale = 1.0 / float(np.sqrt(DH))
  nh2 = H // 2
  wb2 = wb.reshape(1, 3 * D)

  def body(z_ref, wq_ref, wk_ref, wv_ref, bq_ref, bk_ref, bv_ref, o_ref):
    z = z_ref[...]
    q = _dot_nt(z, wq_ref[...]) + bq_ref[...]
    kk = _dot_nt(z, wk_ref[...]) + bk_ref[...]
    v = _dot_nt(z, wv_ref[...]) + bv_ref[...]
    outs = []
    for u in range(2):
      qu = q[:, u * DH:(u + 1) * DH]
      ku = kk[:, u * DH:(u + 1) * DH]
      vu = v[:, u * DH:(u + 1) * DH]
      s_mat = _dot_nt(qu, ku) * scale
      m = jnp.max(s_mat, axis=1, keepdims=True)
      p = jnp.exp(s_mat - m)
      l = jnp.sum(p, axis=1, keepdims=True)
      outs.append(_dot_nn(p / l, vu))
    o_ref[...] = jnp.concatenate(outs, axis=1)

  return pl.pallas_call(
      body,
      grid=(B, nh2),
      in_specs=[
          pl.BlockSpec((S, D), lambda b, h: (b, 0)),
          pl.BlockSpec((2 * DH, D), lambda b, h: (h, 0)),
          pl.BlockSpec((2 * DH, D), lambda b, h: (nh2 + h, 0)),
          pl.BlockSpec((2 * DH, D), lambda b, h: (2 * nh2 + h, 0)),
          pl.BlockSpec((1, 2 * DH), lambda b, h: (0, h)),
          pl.BlockSpec((1, 2 * DH), lambda b, h: (0, nh2 + h)),
          pl.BlockSpec((1, 2 * DH), lambda b, h: (0, 2 * nh2 + h)),
      ],
      out_specs=pl.BlockSpec((S, 2 * DH), lambda b, h: (b, h)),
      out_shape=jax.ShapeDtypeStruct((T, D), jnp.float32),
  )(z, w, w, w, wb2, wb2, wb2)


def _cumsum0(a, n):
  sh = 1
  while sh < n:
    a = a + jnp.concatenate(
        [jnp.zeros((sh, E), jnp.int32), a[:n - sh]], axis=0)
    sh *= 2
  return a


def _k_proj_router(x, ao, w, wb, g, b, sw, sb):
  def body(x_ref, a_ref, w_ref, wb_ref, g_ref, b_ref, sw_ref, sb_ref,
           x1_ref, z_ref, zh_ref, zm_ref, zl_ref, rpm_ref, dst_ref, gi_ref,
           kp_ref, cnt_ref, ps_ref, nd_ref, rpms_ref, rt_sc, rp_sc, rpm_sc):
    t = pl.program_id(0)
    x1 = x_ref[...] + _dot_nt(a_ref[...], w_ref[...]) + wb_ref[...]
    x1_ref[...] = x1
    z = _ln(x1, g_ref[...], b_ref[...])
    z_ref[...] = z
    zh = z.astype(jnp.bfloat16)
    zh_ref[...] = zh
    r1 = z - zh.astype(jnp.float32)
    zm = r1.astype(jnp.bfloat16)
    zm_ref[...] = zm
    zl_ref[...] = (r1 - zm.astype(jnp.float32)).astype(jnp.bfloat16)
    lg = _dot_nn(z, sw_ref[...]) + sb_ref[...]
    mx = jnp.max(lg, axis=1, keepdims=True)
    ex = jnp.exp(lg - mx)
    rp = ex / jnp.sum(ex, axis=1, keepdims=True)
    pm = jnp.max(rp, axis=1, keepdims=True)
    rpm_ref[...] = pm
    ii = lax.broadcasted_iota(jnp.int32, (BT, E), 1)
    rt = jnp.min(jnp.where(rp == pm, ii, E), axis=1, keepdims=True)
    rt_sc[pl.ds(t * BT, BT), :] = rt
    rp_sc[pl.ds(t * BT, BT), :] = rp
    rpm_sc[pl.ds(t * BT, BT), :] = pm

    @pl.when(t == NT - 1)
    def _():
      rte = rt_sc[...]
      ii_s = lax.broadcasted_iota(jnp.int32, (S, E), 1)
      oh0 = (rte[:S, :] == ii_s).astype(jnp.int32)
      oh1 = (rte[S:, :] == ii_s).astype(jnp.int32)
      c0 = _cumsum0(oh0, S)
      c1 = _cumsum0(oh1, S)
      pos0 = jnp.sum((c0 - 1 + c1 - oh1) * oh0, axis=1, keepdims=True)
      pos1 = jnp.sum((c0 + c1 - 1) * oh1, axis=1, keepdims=True)
      pos = jnp.concatenate([pos0, pos1], axis=0)
      keep = (pos < CAP).astype(jnp.int32)
      cnt_ref[...] = (c0[S - 1:S, :] + c1[S - 1:S, :]).astype(jnp.float32)
      ps_ref[...] = jnp.sum(rp_sc[...], axis=0, keepdims=True)
      nd_ref[...] = jnp.sum(1 - keep, axis=0, keepdims=True)
      dst_ref[...] = rte * CPAD + jnp.minimum(pos, CAP)
      gi_ref[...] = rte * CPAD + jnp.minimum(pos, CAP - 1)
      kp_ref[...] = keep
      rpms_ref[...] = jnp.concatenate(
          [rpm_sc[:S, :], rpm_sc[S:, :]], axis=1)

  full = lambda t: (0, 0)
  return pl.pallas_call(
      body,
      grid=(NT,),
      in_specs=[
          pl.BlockSpec((BT, D), lambda t: (t, 0)),
          pl.BlockSpec((BT, D), lambda t: (t, 0)),
          pl.BlockSpec((D, D), full),
          pl.BlockSpec((1, D), full),
          pl.BlockSpec((1, D), full),
          pl.BlockSpec((1, D), full),
          pl.BlockSpec((D, E), full),
          pl.BlockSpec((1, E), full),
      ],
      out_specs=[
          pl.BlockSpec((BT, D), lambda t: (t, 0)),
          pl.BlockSpec((BT, D), lambda t: (t, 0)),
          pl.BlockSpec((BT, D), lambda t: (t, 0)),
          pl.BlockSpec((BT, D), lambda t: (t, 0)),
          pl.BlockSpec((BT, D), lambda t: (t, 0)),
          pl.BlockSpec((BT, 1), lambda t: (t, 0)),
          pl.BlockSpec((T, 1), full),
          pl.BlockSpec((T, 1), full),
          pl.BlockSpec((T, 1), full),
          pl.BlockSpec((1, E), full),
          pl.BlockSpec((1, E), full),
          pl.BlockSpec((1, 1), full),
          pl.BlockSpec((S, B), full),
      ],
      out_shape=[
          jax.ShapeDtypeStruct((T, D), jnp.float32),
          jax.ShapeDtypeStruct((T, D), jnp.float32),
          jax.ShapeDtypeStruct((T, D), jnp.bfloat16),
          jax.ShapeDtypeStruct((T, D), jnp.bfloat16),
          jax.ShapeDtypeStruct((T, D), jnp.bfloat16),
          jax.ShapeDtypeStruct((T, 1), jnp.float32),
          jax.ShapeDtypeStruct((T, 1), jnp.int32),
          jax.ShapeDtypeStruct((T, 1), jnp.int32),
          jax.ShapeDtypeStruct((T, 1), jnp.int32),
          jax.ShapeDtypeStruct((1, E), jnp.float32),
          jax.ShapeDtypeStruct((1, E), jnp.float32),
          jax.ShapeDtypeStruct((1, 1), jnp.int32),
          jax.ShapeDtypeStruct((S, B), jnp.float32),
      ],
      scratch_shapes=[pltpu.VMEM((T, 1), jnp.int32),
                      pltpu.VMEM((T, E), jnp.float32),
                      pltpu.VMEM((T, 1), jnp.float32)],
  )(x, ao, w, wb.reshape(1, D), g.reshape(1, D), b.reshape(1, D), sw,
    sb.reshape(1, E))


def _k_expert(zh, zm, zl, dst, w1, b1, w2, b2, fast):
  nj = DFF // DC
  ddef = lax.Precision.DEFAULT

  def body(zh_ref, zm_ref, zl_ref, dst_ref, w1_ref, b1_ref, w2_ref, b2_ref,
           *out_and_scratch):
    if fast:
      eh_ref, be_s, acc_s = out_and_scratch
      em_ref = el_ref = None
    else:
      eh_ref, em_ref, el_ref, be_s, acc_s = out_and_scratch
    e = pl.program_id(0)
    j = pl.program_id(1)

    @pl.when(j == 0)
    def _():
      slots = e * CPAD + lax.broadcasted_iota(jnp.int32, (T, CPAD), 1)
      pt = (dst_ref[...] == slots).astype(jnp.bfloat16)
      d_tn = lambda a, bb: lax.dot_general(
          a, bb, (((0,), (0,)), ((), ())), precision=ddef,
          preferred_element_type=jnp.float32)
      be = d_tn(pt, zh_ref[...])
      if not fast:
        be = be + (d_tn(pt, zm_ref[...]) + d_tn(pt, zl_ref[...]))
      be_s[...] = be

    if fast:
      bb = be_s[...].astype(jnp.bfloat16)
      h = jnp.maximum(
          _dot_nn(bb, w1_ref[0].astype(jnp.bfloat16), precision=ddef)
          + b1_ref[0], 0.0)
      part = _dot_nn(h.astype(jnp.bfloat16), w2_ref[0].astype(jnp.bfloat16),
                     precision=ddef)
    else:
      h = jnp.maximum(_dot_nn(be_s[...], w1_ref[0]) + b1_ref[0], 0.0)
      part = _dot_nn(h, w2_ref[0])

    @pl.when(j == 0)
    def _():
      acc_s[...] = part + b2_ref[0]

    @pl.when(j != 0)
    def _():
      acc_s[...] = acc_s[...] + part

    @pl.when(j == nj - 1)
    def _():
      eo = acc_s[...]
      eh = eo.astype(jnp.bfloat16)
      eh_ref[...] = eh
      if not fast:
        r1 = eo - eh.astype(jnp.float32)
        em = r1.astype(jnp.bfloat16)
        em_ref[...] = em
        el_ref[...] = (r1 - em.astype(jnp.float32)).astype(jnp.bfloat16)

  eo_shape = jax.ShapeDtypeStruct((E * CPAD, D), jnp.bfloat16)
  n_out = 1 if fast else 3
  out = pl.pallas_call(
      body,
      grid=(E, nj),
      in_specs=[
          pl.BlockSpec((T, D), lambda e, j: (0, 0)),
          pl.BlockSpec((T, D), lambda e, j: (0, 0)),
          pl.BlockSpec((T, D), lambda e, j: (0, 0)),
          pl.BlockSpec((T, 1), lambda e, j: (0, 0)),
          pl.BlockSpec((1, D, DC), lambda e, j: (e, 0, j)),
          pl.BlockSpec((1, 1, DC), lambda e, j: (e, 0, j)),
          pl.BlockSpec((1, DC, D), lambda e, j: (e, j, 0)),
          pl.BlockSpec((1, 1, D), lambda e, j: (e, 0, 0)),
      ],
      out_specs=[pl.BlockSpec((CPAD, D), lambda e, j: (e, 0))] * n_out,
      out_shape=[eo_shape] * n_out,
      scratch_shapes=[pltpu.VMEM((CPAD, D), jnp.float32),
                      pltpu.VMEM((CPAD, D), jnp.float32)],
  )(zh, zm, zl, dst, w1, b1.reshape(E, 1, DFF), w2, b2.reshape(E, 1, D))
  return (out[0], None, None) if fast else tuple(out)


def _gather_eo(gi_blk, eh_ref, em_ref, el_ref):
  ddef = lax.Precision.DEFAULT
  slots = lax.broadcasted_iota(jnp.int32, (BT, E * CPAD), 1)
  g_1h = (slots == gi_blk).astype(jnp.bfloat16)
  gat = _dot_nn(g_1h, eh_ref[...], precision=ddef)
  if em_ref is not None:
    gat = gat + (_dot_nn(g_1h, em_ref[...], precision=ddef) +
                 _dot_nn(g_1h, el_ref[...], precision=ddef))
  return gat


def _k_combine_ln(x1, z, eh, em, el, gi, kp, rpm, g, b):
  def body(x_ref, z_ref, eh_ref, em_ref, el_ref, gi_ref, k_ref, p_ref,
           g_ref, b_ref, o_ref, zn_ref):
    gat = _gather_eo(gi_ref[...], eh_ref, em_ref, el_ref)
    sel = jnp.where(k_ref[...] != 0, gat, z_ref[...])
    x2 = x_ref[...] + sel * p_ref[...]
    o_ref[...] = x2
    zn_ref[...] = _ln(x2, g_ref[...], b_ref[...])

  return pl.pallas_call(
      body,
      grid=(NT,),
      in_specs=[
          pl.BlockSpec((BT, D), lambda t: (t, 0)),
          pl.BlockSpec((BT, D), lambda t: (t, 0)),
          pl.BlockSpec((E * CPAD, D), lambda t: (0, 0)),
          pl.BlockSpec((E * CPAD, D), lambda t: (0, 0)),
          pl.BlockSpec((E * CPAD, D), lambda t: (0, 0)),
          pl.BlockSpec((BT, 1), lambda t: (t, 0)),
          pl.BlockSpec((BT, 1), lambda t: (t, 0)),
          pl.BlockSpec((BT, 1), lambda t: (t, 0)),
          pl.BlockSpec((1, D), lambda t: (0, 0)),
          pl.BlockSpec((1, D), lambda t: (0, 0)),
      ],
      out_specs=[pl.BlockSpec((BT, D), lambda t: (t, 0)),
                 pl.BlockSpec((BT, D), lambda t: (t, 0))],
      out_shape=[jax.ShapeDtypeStruct((T, D), jnp.float32),
                 jax.ShapeDtypeStruct((T, D), jnp.float32)],
  )(x1, z, eh, em, el, gi, kp, rpm, g.reshape(1, D), b.reshape(1, D))


def _k_combine_fln(x1, z, eh, gi, kp, rpm, g, b):
  def body(x_ref, z_ref, eh_ref, gi_ref, k_ref, p_ref, g_ref, b_ref, o_ref):
    gat = _gather_eo(gi_ref[...], eh_ref, None, None)
    sel = jnp.where(k_ref[...] != 0, gat, z_ref[...])
    xf = x_ref[...] + sel * p_ref[...]
    o_ref[...] = _ln(xf, g_ref[...], b_ref[...]).astype(jnp.bfloat16)

  return pl.pallas_call(
      body,
      grid=(NT,),
      in_specs=[
          pl.BlockSpec((BT, D), lambda t: (t, 0)),
          pl.BlockSpec((BT, D), lambda t: (t, 0)),
          pl.BlockSpec((E * CPAD, D), lambda t: (0, 0)),
          pl.BlockSpec((BT, 1), lambda t: (t, 0)),
          pl.BlockSpec((BT, 1), lambda t: (t, 0)),
          pl.BlockSpec((BT, 1), lambda t: (t, 0)),
          pl.BlockSpec((1, D), lambda t: (0, 0)),
          pl.BlockSpec((1, D), lambda t: (0, 0)),
      ],
      out_specs=pl.BlockSpec((BT, D), lambda t: (t, 0)),
      out_shape=jax.ShapeDtypeStruct((T, D), jnp.bfloat16),
  )(x1, z, eh, gi, kp, rpm, g.reshape(1, D), b.reshape(1, D))


def _k_dec_lse(xb, w, db):
  def body(x_ref, w_ref, db_ref, lse_ref, m_acc, s_acc):
    v = pl.program_id(0)

    @pl.when(v == 0)
    def _():
      m_acc[...] = jnp.full((T, 1), -1e30, jnp.float32)
      s_acc[...] = jnp.zeros((T, 1), jnp.float32)

    wb = w_ref[...].astype(jnp.bfloat16)
    for c in range(NT):
      xc = x_ref[pl.ds(c * BT, BT), :]
      lg = lax.dot_general(xc, wb, (((1,), (1,)), ((), ())),
                           preferred_element_type=jnp.float32) + db_ref[...]
      mo = m_acc[pl.ds(c * BT, BT), :]
      so = s_acc[pl.ds(c * BT, BT), :]
      tm = jnp.max(lg, axis=1, keepdims=True)
      mn = jnp.maximum(mo, tm)
      sn = so * jnp.exp(mo - mn) + jnp.sum(jnp.exp(lg - mn), axis=1,
                                           keepdims=True)
      m_acc[pl.ds(c * BT, BT), :] = mn
      s_acc[pl.ds(c * BT, BT), :] = sn

    @pl.when(v == NV - 1)
    def _():
      lse_ref[...] = m_acc[...] + jnp.log(s_acc[...])

  return pl.pallas_call(
      body,
      grid=(NV,),
      in_specs=[
          pl.BlockSpec((T, D), lambda v: (0, 0)),
          pl.BlockSpec((VT, D), lambda v: (v, 0)),
          pl.BlockSpec((1, VT), lambda v: (0, v)),
      ],
      out_specs=pl.BlockSpec((T, 1), lambda v: (0, 0)),
      out_shape=jax.ShapeDtypeStruct((T, 1), jnp.float32),
      scratch_shapes=[pltpu.VMEM((T, 1), jnp.float32),
                      pltpu.VMEM((T, 1), jnp.float32)],
  )(xb, w, db)


def _k_dec_out(xb, w, db, lse):
  def body(x_ref, w_ref, db_ref, l_ref, o_ref, wb_s):
    t = pl.program_id(1)

    @pl.when(t == 0)
    def _():
      wb_s[...] = w_ref[...].astype(jnp.bfloat16)

    xc = x_ref[pl.ds(t * BT, BT), :]
    lg = lax.dot_general(xc, wb_s[...], (((1,), (1,)), ((), ())),
                         preferred_element_type=jnp.float32) + db_ref[...]
    o_ref[...] = lg - l_ref[pl.ds(t * BT, BT), :]

  return pl.pallas_call(
      body,
      grid=(NV, NT),
      in_specs=[
          pl.BlockSpec((T, D), lambda v, t: (0, 0)),
          pl.BlockSpec((VT, D), lambda v, t: (v, 0)),
          pl.BlockSpec((1, VT), lambda v, t: (0, v)),
          pl.BlockSpec((T, 1), lambda v, t: (0, 0)),
      ],
      out_specs=pl.BlockSpec((BT, VT), lambda v, t: (t, v)),
      out_shape=jax.ShapeDtypeStruct((T, VOCAB), jnp.float32),
      scratch_shapes=[pltpu.VMEM((VT, D), jnp.bfloat16)],
  )(xb, w, db, lse)


def kernel(input_chars, embed_w, ln1_g, ln1_b, ln2_g, ln2_b, attn_in_w,
           attn_in_b, attn_out_w, attn_out_b, switch_w, switch_b, exp_w1,
           exp_b1, exp_w2, exp_b2, fln_g, fln_b, dec_w, dec_b):
  idx = input_chars.reshape(T)
  emb = _sc_gather_rows(embed_w, idx)
  x, zatt = _k_addpe_ln(emb, jnp.asarray(_pe_np()), ln1_g[0], ln1_b[0])
  counts_l, ps_l, nd_l, rpm_l = [], [], [], []
  xb = None
  for i in range(2):
    fast = i == 1
    ao = _k_attn_fused(zatt, attn_in_w[i], attn_in_b[i])
    (x1, z, zh, zm, zl, rpm, dst, gi, kp, cnt, ps, nd,
     rpm_s) = _k_proj_router(
        x, ao, attn_out_w[i], attn_out_b[i], ln2_g[i], ln2_b[i], switch_w[i],
        switch_b[i])
    eh, em, el = _k_expert(zh, zm, zl, dst, exp_w1[i], exp_b1[i], exp_w2[i],
                           exp_b2[i], fast=fast)
    if not fast:
      x, zatt = _k_combine_ln(x1, z, eh, em, el, gi, kp, rpm, ln1_g[i + 1],
                              ln1_b[i + 1])
    else:
      xb = _k_combine_fln(x1, z, eh, gi, kp, rpm, fln_g, fln_b)
    counts_l.append(cnt.reshape(E))
    ps_l.append(ps.reshape(E))
    nd_l.append(nd.reshape(()))
    rpm_l.append(rpm_s.reshape(T))
  db2 = dec_b.reshape(1, VOCAB)
  lse = _k_dec_lse(xb, dec_w, db2)
  lp = _k_dec_out(xb, dec_w, db2, lse)
  return (lp.reshape(B, S, VOCAB), jnp.stack(counts_l), jnp.stack(ps_l),
          jnp.stack(nd_l), jnp.stack(rpm_l))


# speedup vs baseline: 1.3492x; 1.0051x over previous
import functools

import numpy as np
import jax
import jax.numpy as jnp
from jax import lax
from jax.experimental import pallas as pl
from jax.experimental.pallas import tpu as pltpu
from jax.experimental.pallas import tpu_sc as plsc

B = 2
S = 1024
D = 1024
H = 16
DH = D // H
E = 16
DFF = 2048
T = B * S
CAP = int(1.2 * T / E)
CPAD = 160
VOCAB = 32000
BT = 256
NT = T // BT
VT = 3200
NV = VOCAB // VT
DC = 512
_NW = 32

def _split_hl(a):
  ah = a.astype(jnp.bfloat16)
  al = (a - ah.astype(jnp.float32)).astype(jnp.bfloat16)
  return ah, al


def _dot_nt(a, b, precision=None):
  dims = (((1,), (1,)), ((), ()))
  if precision is not None:
    return lax.dot_general(a, b, dims, precision=precision,
                           preferred_element_type=jnp.float32)
  ah, al = _split_hl(a)
  bh, bl = _split_hl(b)
  d = lambda x, y: lax.dot_general(x, y, dims,
                                   preferred_element_type=jnp.float32)
  return d(ah, bh) + (d(ah, bl) + d(al, bh))


def _dot_nn(a, b, precision=None):
  dims = (((1,), (0,)), ((), ()))
  if precision is not None:
    return lax.dot_general(a, b, dims, precision=precision,
                           preferred_element_type=jnp.float32)
  ah, al = _split_hl(a)
  bh, bl = _split_hl(b)
  d = lambda x, y: lax.dot_general(x, y, dims,
                                   preferred_element_type=jnp.float32)
  return d(ah, bh) + (d(ah, bl) + d(al, bh))


def _ln(x, g, b):
  m = jnp.mean(x, axis=-1, keepdims=True)
  v = jnp.mean((x - m) ** 2, axis=-1, keepdims=True)
  return (x - m) / jnp.sqrt(v + 1e-5) * g + b


def _pe_np():
  pos = np.arange(S, dtype=np.float32)[:, None]
  div = np.exp(np.arange(0, D, 2, dtype=np.float32) * (-np.log(10000.0) / D))
  pe = np.zeros((S, D), dtype=np.float32)
  pe[:, 0::2] = np.sin(pos * div)
  pe[:, 1::2] = np.cos(pos * div)
  return pe


def _sc_gather_rows(table, idx):
  t = idx.shape[0]
  d = table.shape[1]
  bpw = t // _NW
  idx2 = idx.reshape(_NW, bpw)
  mesh = plsc.VectorSubcoreMesh(core_axis_name="c", subcore_axis_name="s")

  @functools.partial(
      pl.kernel, mesh=mesh,
      out_type=jax.ShapeDtypeStruct((t, d), jnp.float32),
      scratch_types=[
          pltpu.VMEM((bpw,), jnp.int32),
          pltpu.VMEM((bpw, d), jnp.float32),
          pltpu.SemaphoreType.DMA,
      ],
  )
  def k(table_hbm, idx_hbm, out_hbm, idx_v, rows_v, sem):
    wid = lax.axis_index("s") * 2 + lax.axis_index("c")
    pltpu.sync_copy(idx_hbm.at[wid], idx_v)
    pltpu.async_copy(table_hbm.at[idx_v], rows_v, sem).wait()
    pltpu.sync_copy(rows_v, out_hbm.at[pl.ds(wid * bpw, bpw)])

  return k(table, idx2)


def _k_addpe_ln(emb, pe, g, b):
  def body(e_ref, p_ref, g_ref, b_ref, x_ref, z_ref):
    x = e_ref[...] + p_ref[...]
    x_ref[...] = x
    z_ref[...] = _ln(x, g_ref[...], b_ref[...])

  return pl.pallas_call(
      body,
      grid=(NT,),
      in_specs=[pl.BlockSpec((BT, D), lambda i: (i, 0)),
                pl.BlockSpec((BT, D), lambda i: (i % (S // BT), 0)),
                pl.BlockSpec((1, D), lambda i: (0, 0)),
                pl.BlockSpec((1, D), lambda i: (0, 0))],
      out_specs=[pl.BlockSpec((BT, D), lambda i: (i, 0)),
                 pl.BlockSpec((BT, D), lambda i: (i, 0))],
      out_shape=[jax.ShapeDtypeStruct((T, D), jnp.float32),
                 jax.ShapeDtypeStruct((T, D), jnp.float32)],
  )(emb, pe, g.reshape(1, D), b.reshape(1, D))


def _k_attn_fused(z, w, wb):
  scale = 1.0 / float(np.sqrt(DH))
  nh2 = H // 2
  wb2 = wb.reshape(1, 3 * D)

  def body(z_ref, wq_ref, wk_ref, wv_ref, bq_ref, bk_ref, bv_ref, o_ref):
    z = z_ref[...]
    q = _dot_nt(z, wq_ref[...]) + bq_ref[...]
    kk = _dot_nt(z, wk_ref[...]) + bk_ref[...]
    v = _dot_nt(z, wv_ref[...]) + bv_ref[...]
    outs = []
    for u in range(2):
      qu = q[:, u * DH:(u + 1) * DH]
      ku = kk[:, u * DH:(u + 1) * DH]
      vu = v[:, u * DH:(u + 1) * DH]
      s_mat = _dot_nt(qu, ku) * scale
      m = jnp.max(s_mat, axis=1, keepdims=True)
      p = jnp.exp(s_mat - m)
      l = jnp.sum(p, axis=1, keepdims=True)
      outs.append(_dot_nn(p / l, vu))
    o_ref[...] = jnp.concatenate(outs, axis=1)

  return pl.pallas_call(
      body,
      grid=(B, nh2),
      in_specs=[
          pl.BlockSpec((S, D), lambda b, h: (b, 0)),
          pl.BlockSpec((2 * DH, D), lambda b, h: (h, 0)),
          pl.BlockSpec((2 * DH, D), lambda b, h: (nh2 + h, 0)),
          pl.BlockSpec((2 * DH, D), lambda b, h: (2 * nh2 + h, 0)),
          pl.BlockSpec((1, 2 * DH), lambda b, h: (0, h)),
          pl.BlockSpec((1, 2 * DH), lambda b, h: (0, nh2 + h)),
          pl.BlockSpec((1, 2 * DH), lambda b, h: (0, 2 * nh2 + h)),
      ],
      out_specs=pl.BlockSpec((S, 2 * DH), lambda b, h: (b, h)),
      out_shape=jax.ShapeDtypeStruct((T, D), jnp.float32),
  )(z, w, w, w, wb2, wb2, wb2)


def _cumsum0(a, n):
  sh = 1
  while sh < n:
    a = a + jnp.concatenate(
        [jnp.zeros((sh, E), jnp.int32), a[:n - sh]], axis=0)
    sh *= 2
  return a


def _k_proj_router(x, ao, w, wb, g, b, sw, sb):
  def body(x_ref, a_ref, w_ref, wb_ref, g_ref, b_ref, sw_ref, sb_ref,
           x1_ref, zh_ref, zm_ref, zl_ref, rpm_ref, dst_ref, gi_ref,
           kp_ref, cnt_ref, ps_ref, nd_ref, rpms_ref, rt_sc, rp_sc, rpm_sc):
    t = pl.program_id(0)
    x1 = x_ref[...] + _dot_nt(a_ref[...], w_ref[...]) + wb_ref[...]
    x1_ref[...] = x1
    z = _ln(x1, g_ref[...], b_ref[...])
    zh = z.astype(jnp.bfloat16)
    zh_ref[...] = zh
    r1 = z - zh.astype(jnp.float32)
    zm = r1.astype(jnp.bfloat16)
    zm_ref[...] = zm
    zl_ref[...] = (r1 - zm.astype(jnp.float32)).astype(jnp.bfloat16)
    lg = _dot_nn(z, sw_ref[...]) + sb_ref[...]
    mx = jnp.max(lg, axis=1, keepdims=True)
    ex = jnp.exp(lg - mx)
    rp = ex / jnp.sum(ex, axis=1, keepdims=True)
    pm = jnp.max(rp, axis=1, keepdims=True)
    rpm_ref[...] = pm
    ii = lax.broadcasted_iota(jnp.int32, (BT, E), 1)
    rt = jnp.min(jnp.where(rp == pm, ii, E), axis=1, keepdims=True)
    rt_sc[pl.ds(t * BT, BT), :] = rt
    rp_sc[pl.ds(t * BT, BT), :] = rp
    rpm_sc[pl.ds(t * BT, BT), :] = pm

    @pl.when(t == NT - 1)
    def _():
      rte = rt_sc[...]
      ii_s = lax.broadcasted_iota(jnp.int32, (S, E), 1)
      oh0 = (rte[:S, :] == ii_s).astype(jnp.int32)
      oh1 = (rte[S:, :] == ii_s).astype(jnp.int32)
      c0 = _cumsum0(oh0, S)
      c1 = _cumsum0(oh1, S)
      pos0 = jnp.sum((c0 - 1 + c1 - oh1) * oh0, axis=1, keepdims=True)
      pos1 = jnp.sum((c0 + c1 - 1) * oh1, axis=1, keepdims=True)
      pos = jnp.concatenate([pos0, pos1], axis=0)
      keep = (pos < CAP).astype(jnp.int32)
      cnt_ref[...] = (c0[S - 1:S, :] + c1[S - 1:S, :]).astype(jnp.float32)
      ps_ref[...] = jnp.sum(rp_sc[...], axis=0, keepdims=True)
      nd_ref[...] = jnp.sum(1 - keep, axis=0, keepdims=True)
      dst_ref[...] = rte * CPAD + jnp.minimum(pos, CAP)
      gi_ref[...] = rte * CPAD + jnp.minimum(pos, CAP - 1)
      kp_ref[...] = keep
      rpms_ref[...] = jnp.concatenate(
          [rpm_sc[:S, :], rpm_sc[S:, :]], axis=1)

  full = lambda t: (0, 0)
  return pl.pallas_call(
      body,
      grid=(NT,),
      in_specs=[
          pl.BlockSpec((BT, D), lambda t: (t, 0)),
          pl.BlockSpec((BT, D), lambda t: (t, 0)),
          pl.BlockSpec((D, D), full),
          pl.BlockSpec((1, D), full),
          pl.BlockSpec((1, D), full),
          pl.BlockSpec((1, D), full),
          pl.BlockSpec((D, E), full),
          pl.BlockSpec((1, E), full),
      ],
      out_specs=[
          pl.BlockSpec((BT, D), lambda t: (t, 0)),
          pl.BlockSpec((BT, D), lambda t: (t, 0)),
          pl.BlockSpec((BT, D), lambda t: (t, 0)),
          pl.BlockSpec((BT, D), lambda t: (t, 0)),
          pl.BlockSpec((BT, 1), lambda t: (t, 0)),
          pl.BlockSpec((T, 1), full),
          pl.BlockSpec((T, 1), full),
          pl.BlockSpec((T, 1), full),
          pl.BlockSpec((1, E), full),
          pl.BlockSpec((1, E), full),
          pl.BlockSpec((1, 1), full),
          pl.BlockSpec((S, B), full),
      ],
      out_shape=[
          jax.ShapeDtypeStruct((T, D), jnp.float32),
          jax.ShapeDtypeStruct((T, D), jnp.bfloat16),
          jax.ShapeDtypeStruct((T, D), jnp.bfloat16),
          jax.ShapeDtypeStruct((T, D), jnp.bfloat16),
          jax.ShapeDtypeStruct((T, 1), jnp.float32),
          jax.ShapeDtypeStruct((T, 1), jnp.int32),
          jax.ShapeDtypeStruct((T, 1), jnp.int32),
          jax.ShapeDtypeStruct((T, 1), jnp.int32),
          jax.ShapeDtypeStruct((1, E), jnp.float32),
          jax.ShapeDtypeStruct((1, E), jnp.float32),
          jax.ShapeDtypeStruct((1, 1), jnp.int32),
          jax.ShapeDtypeStruct((S, B), jnp.float32),
      ],
      scratch_shapes=[pltpu.VMEM((T, 1), jnp.int32),
                      pltpu.VMEM((T, E), jnp.float32),
                      pltpu.VMEM((T, 1), jnp.float32)],
  )(x, ao, w, wb.reshape(1, D), g.reshape(1, D), b.reshape(1, D), sw,
    sb.reshape(1, E))


def _k_expert_combine(zh, zm, zl, dst, w1, b1, w2, b2, x1, gi, kp, rpm,
                      g, b, fast):
  nj = DFF // DC
  nexp = E * nj
  ddef = lax.Precision.DEFAULT

  def body(zh_ref, zm_ref, zl_ref, dst_ref, w1_ref, b1_ref, w2_ref, b2_ref,
           x1_ref, gi_ref, kp_ref, rpm_ref, g_ref, b_ref,
           x2_ref, zn_ref, be_s, acc_s, eh_s, em_s, el_s):
    pid = pl.program_id(0)
    ecl = jnp.minimum(pid, nexp - 1)
    e = ecl // nj
    j = ecl % nj

    @pl.when(pid < nexp)
    def _expert():
      @pl.when(j == 0)
      def _():
        slots = e * CPAD + lax.broadcasted_iota(jnp.int32, (T, CPAD), 1)
        pt = (dst_ref[...] == slots).astype(jnp.bfloat16)
        d_tn = lambda a, bb: lax.dot_general(
            a, bb, (((0,), (0,)), ((), ())), precision=ddef,
            preferred_element_type=jnp.float32)
        be = d_tn(pt, zh_ref[...])
        if not fast:
          be = be + (d_tn(pt, zm_ref[...]) + d_tn(pt, zl_ref[...]))
        be_s[...] = be

      if fast:
        bb = be_s[...].astype(jnp.bfloat16)
        h = jnp.maximum(
            _dot_nn(bb, w1_ref[0].astype(jnp.bfloat16), precision=ddef)
            + b1_ref[0], 0.0)
        part = _dot_nn(h.astype(jnp.bfloat16), w2_ref[0].astype(jnp.bfloat16),
                       precision=ddef)
      else:
        h = jnp.maximum(_dot_nn(be_s[...], w1_ref[0]) + b1_ref[0], 0.0)
        part = _dot_nn(h, w2_ref[0])

      @pl.when(j == 0)
      def _():
        acc_s[...] = part + b2_ref[0]

      @pl.when(j != 0)
      def _():
        acc_s[...] = acc_s[...] + part

      @pl.when(j == nj - 1)
      def _():
        eo = acc_s[...]
        eh = eo.astype(jnp.bfloat16)
        eh_s[pl.ds(e * CPAD, CPAD), :] = eh
        if not fast:
          r1 = eo - eh.astype(jnp.float32)
          em = r1.astype(jnp.bfloat16)
          em_s[pl.ds(e * CPAD, CPAD), :] = em
          el_s[pl.ds(e * CPAD, CPAD), :] = (
              r1 - em.astype(jnp.float32)).astype(jnp.bfloat16)

    @pl.when(pid >= nexp)
    def _combine():
      t = pid - nexp
      gat = _gather_eo(gi_ref[...], eh_s, em_s if not fast else None,
                       el_s if not fast else None)
      tok = pl.ds(t * BT, BT)
      z = zh_ref[tok, :].astype(jnp.float32)
      if not fast:
        z = z + (zm_ref[tok, :].astype(jnp.float32) +
                 zl_ref[tok, :].astype(jnp.float32))
      sel = jnp.where(kp_ref[...] != 0, gat, z)
      x2 = x1_ref[...] + sel * rpm_ref[...]
      zn = _ln(x2, g_ref[...], b_ref[...])
      if fast:
        zn_ref[...] = zn.astype(jnp.bfloat16)
      else:
        x2_ref[...] = x2
        zn_ref[...] = zn

  tmap = lambda p: (jnp.maximum(p - nexp, 0), 0)
  full = lambda p: (0, 0)
  out_specs = [pl.BlockSpec((BT, D), tmap)]
  out_shape = [jax.ShapeDtypeStruct((T, D), jnp.bfloat16 if fast
                                    else jnp.float32)]
  if not fast:
    out_specs = [pl.BlockSpec((BT, D), tmap)] + out_specs
    out_shape = [jax.ShapeDtypeStruct((T, D), jnp.float32)] + out_shape
  eo_sd = pltpu.VMEM((E * CPAD, D), jnp.bfloat16)
  out = pl.pallas_call(
      body if not fast else (lambda *a: body(*a[:14], None, *a[14:])),
      grid=(nexp + NT,),
      in_specs=[
          pl.BlockSpec((T, D), full),
          pl.BlockSpec((T, D), full),
          pl.BlockSpec((T, D), full),
          pl.BlockSpec((T, 1), full),
          pl.BlockSpec((1, D, DC), lambda p: (jnp.minimum(p, nexp - 1) // nj,
                                              0,
                                              jnp.minimum(p, nexp - 1) % nj)),
          pl.BlockSpec((1, 1, DC), lambda p: (jnp.minimum(p, nexp - 1) // nj,
                                              0,
                                              jnp.minimum(p, nexp - 1) % nj)),
          pl.BlockSpec((1, DC, D), lambda p: (jnp.minimum(p, nexp - 1) // nj,
                                              jnp.minimum(p, nexp - 1) % nj,
                                              0)),
          pl.BlockSpec((1, 1, D), lambda p: (jnp.minimum(p, nexp - 1) // nj,
                                             0, 0)),
          pl.BlockSpec((BT, D), tmap),
          pl.BlockSpec((BT, 1), tmap),
          pl.BlockSpec((BT, 1), tmap),
          pl.BlockSpec((BT, 1), tmap),
          pl.BlockSpec((1, D), full),
          pl.BlockSpec((1, D), full),
      ],
      out_specs=out_specs,
      out_shape=out_shape,
      scratch_shapes=[pltpu.VMEM((CPAD, D), jnp.float32),
                      pltpu.VMEM((CPAD, D), jnp.float32),
                      eo_sd, eo_sd, eo_sd],
  )(zh, zm, zl, dst, w1, b1.reshape(E, 1, DFF), w2, b2.reshape(E, 1, D),
    x1, gi, kp, rpm, g.reshape(1, D), b.reshape(1, D))
  return out if not fast else (None, out[0])


def _gather_eo(gi_blk, eh_ref, em_ref, el_ref):
  ddef = lax.Precision.DEFAULT
  slots = lax.broadcasted_iota(jnp.int32, (BT, E * CPAD), 1)
  g_1h = (slots == gi_blk).astype(jnp.bfloat16)
  gat = _dot_nn(g_1h, eh_ref[...], precision=ddef)
  if em_ref is not None:
    gat = gat + (_dot_nn(g_1h, em_ref[...], precision=ddef) +
                 _dot_nn(g_1h, el_ref[...], precision=ddef))
  return gat


def _k_dec_lse(xb, w, db):
  def body(x_ref, w_ref, db_ref, lse_ref, m_acc, s_acc):
    v = pl.program_id(0)

    @pl.when(v == 0)
    def _():
      m_acc[...] = jnp.full((T, 1), -1e30, jnp.float32)
      s_acc[...] = jnp.zeros((T, 1), jnp.float32)

    wb = w_ref[...].astype(jnp.bfloat16)
    for c in range(NT):
      xc = x_ref[pl.ds(c * BT, BT), :]
      lg = lax.dot_general(xc, wb, (((1,), (1,)), ((), ())),
                           preferred_element_type=jnp.float32) + db_ref[...]
      mo = m_acc[pl.ds(c * BT, BT), :]
      so = s_acc[pl.ds(c * BT, BT), :]
      tm = jnp.max(lg, axis=1, keepdims=True)
      mn = jnp.maximum(mo, tm)
      sn = so * jnp.exp(mo - mn) + jnp.sum(jnp.exp(lg - mn), axis=1,
                                           keepdims=True)
      m_acc[pl.ds(c * BT, BT), :] = mn
      s_acc[pl.ds(c * BT, BT), :] = sn

    @pl.when(v == NV - 1)
    def _():
      lse_ref[...] = m_acc[...] + jnp.log(s_acc[...])

  return pl.pallas_call(
      body,
      grid=(NV,),
      in_specs=[
          pl.BlockSpec((T, D), lambda v: (0, 0)),
          pl.BlockSpec((VT, D), lambda v: (v, 0)),
          pl.BlockSpec((1, VT), lambda v: (0, v)),
      ],
      out_specs=pl.BlockSpec((T, 1), lambda v: (0, 0)),
      out_shape=jax.ShapeDtypeStruct((T, 1), jnp.float32),
      scratch_shapes=[pltpu.VMEM((T, 1), jnp.float32),
                      pltpu.VMEM((T, 1), jnp.float32)],
  )(xb, w, db)


def _k_dec_out(xb, w, db, lse):
  def body(x_ref, w_ref, db_ref, l_ref, o_ref, wb_s):
    t = pl.program_id(1)

    @pl.when(t == 0)
    def _():
      wb_s[...] = w_ref[...].astype(jnp.bfloat16)

    xc = x_ref[pl.ds(t * BT, BT), :]
    lg = lax.dot_general(xc, wb_s[...], (((1,), (1,)), ((), ())),
                         preferred_element_type=jnp.float32) + db_ref[...]
    o_ref[...] = lg - l_ref[pl.ds(t * BT, BT), :]

  return pl.pallas_call(
      body,
      grid=(NV, NT),
      in_specs=[
          pl.BlockSpec((T, D), lambda v, t: (0, 0)),
          pl.BlockSpec((VT, D), lambda v, t: (v, 0)),
          pl.BlockSpec((1, VT), lambda v, t: (0, v)),
          pl.BlockSpec((T, 1), lambda v, t: (0, 0)),
      ],
      out_specs=pl.BlockSpec((BT, VT), lambda v, t: (t, v)),
      out_shape=jax.ShapeDtypeStruct((T, VOCAB), jnp.float32),
      scratch_shapes=[pltpu.VMEM((VT, D), jnp.bfloat16)],
  )(xb, w, db, lse)


def kernel(input_chars, embed_w, ln1_g, ln1_b, ln2_g, ln2_b, attn_in_w,
           attn_in_b, attn_out_w, attn_out_b, switch_w, switch_b, exp_w1,
           exp_b1, exp_w2, exp_b2, fln_g, fln_b, dec_w, dec_b):
  idx = input_chars.reshape(T)
  emb = _sc_gather_rows(embed_w, idx)
  x, zatt = _k_addpe_ln(emb, jnp.asarray(_pe_np()), ln1_g[0], ln1_b[0])
  counts_l, ps_l, nd_l, rpm_l = [], [], [], []
  xb = None
  for i in range(2):
    fast = i == 1
    ao = _k_attn_fused(zatt, attn_in_w[i], attn_in_b[i])
    (x1, zh, zm, zl, rpm, dst, gi, kp, cnt, ps, nd,
     rpm_s) = _k_proj_router(
        x, ao, attn_out_w[i], attn_out_b[i], ln2_g[i], ln2_b[i], switch_w[i],
        switch_b[i])
    gln = (fln_g, fln_b) if fast else (ln1_g[i + 1], ln1_b[i + 1])
    x, znext = _k_expert_combine(
        zh, zm, zl, dst, exp_w1[i], exp_b1[i], exp_w2[i], exp_b2[i], x1, gi,
        kp, rpm, gln[0], gln[1], fast=fast)
    if not fast:
      zatt = znext
    else:
      xb = znext
    counts_l.append(cnt.reshape(E))
    ps_l.append(ps.reshape(E))
    nd_l.append(nd.reshape(()))
    rpm_l.append(rpm_s.reshape(T))
  db2 = dec_b.reshape(1, VOCAB)
  lse = _k_dec_lse(xb, dec_w, db2)
  lp = _k_dec_out(xb, dec_w, db2, lse)
  return (lp.reshape(B, S, VOCAB), jnp.stack(counts_l), jnp.stack(ps_l),
          jnp.stack(nd_l), jnp.stack(rpm_l))
